# Initial kernel scaffold; baseline (speedup 1.0000x reference)
import jax
import jax.numpy as jnp
from jax.experimental import pallas as pl


def kernel(x, task_bh, params):
    raise NotImplementedError("write your pallas kernel here")


# speedup vs baseline: 2.7800x; 2.7800x over previous
import jax
import jax.numpy as jnp
from jax.experimental import pallas as pl
from jax.experimental.pallas import tpu as pltpu

HIGH = jax.lax.Precision.HIGHEST
F32 = jnp.float32
BF16 = jnp.bfloat16


def _silu(x):
    return x * jax.nn.sigmoid(x)


def _ln_in(x, g, b):
    m = jnp.mean(x, -1, keepdims=True)
    v = jnp.mean((x - m) ** 2, -1, keepdims=True)
    return (x - m) * jax.lax.rsqrt(v + 1e-5) * g + b


def _stage_a_kernel(xp_ref, wk_ref, g_ref, b_ref, w1_ref, o_ref, *, rb, W, Cin, Cd):
    i = pl.program_id(1)
    rows = xp_ref[0, pl.ds(i * rb, rb + 2)]
    acc = jnp.zeros((rb * W, Cin), F32)
    for dy in range(3):
        for dx in range(3):
            xs = rows[dy:dy + rb, dx:dx + W, :].reshape(rb * W, Cin)
            acc = acc + jnp.dot(xs, wk_ref[3 * dy + dx], precision=HIGH)
    y = _silu(acc * g_ref[...] + b_ref[...])
    out = jnp.dot(y, w1_ref[...], precision=HIGH)
    o_ref[0] = out.reshape(rb, W, Cd)


def _gating_kernel(x_ref, wg_ref, gates_ref, loss_ref, imp_ref, load_ref, *, E, nblk):
    t = pl.program_id(0)
    x = x_ref[...]
    logits = jnp.dot(x, wg_ref[...], precision=HIGH)
    tb = logits.shape[0]
    iota = jax.lax.broadcasted_iota(jnp.int32, (tb, E), 1)
    m1 = jnp.max(logits, axis=1, keepdims=True)
    i1 = jnp.argmax(logits, axis=1)[:, None]
    masked = jnp.where(iota == i1, -jnp.inf, logits)
    m2 = jnp.max(masked, axis=1, keepdims=True)
    i2 = jnp.argmax(masked, axis=1)[:, None]
    w1s = jax.nn.sigmoid(m1 - m2)
    w2s = jax.nn.sigmoid(m2 - m1)
    g = jnp.where(iota == i1, w1s, 0.0) + jnp.where(iota == i2, w2s, 0.0)
    gates_ref[...] = g

    @pl.when(t == 0)
    def _():
        imp_ref[...] = jnp.zeros_like(imp_ref)
        load_ref[...] = jnp.zeros_like(load_ref)

    imp_ref[...] += jnp.sum(g, axis=0, keepdims=True)
    load_ref[...] += jnp.sum((g > 0).astype(F32), axis=0, keepdims=True)

    @pl.when(t == nblk - 1)
    def _():
        def cv2(v):
            m = jnp.mean(v)
            var = jnp.mean((v - m) ** 2)
            return var / (m * m + 1e-10)

        val = cv2(imp_ref[0]) + cv2(load_ref[0])
        loss_ref[...] = jnp.broadcast_to(val, (1, 1))


def _moe_kernel(x_ref, g_ref, w1_ref, b1_ref, w2_ref, b2_ref, o_ref):
    e = pl.program_id(1)
    x = x_ref[...]
    h = jnp.dot(x, w1_ref[0], preferred_element_type=F32) + b1_ref[0]
    h = jnp.maximum(h, 0.0).astype(BF16)
    oe = jnp.dot(h, w2_ref[0], preferred_element_type=F32) + b2_ref[0]
    g8 = g_ref[...]
    iota = jax.lax.broadcasted_iota(jnp.int32, g8.shape, 1)
    gcol = jnp.sum(jnp.where(iota == e, g8, 0.0), axis=1, keepdims=True)
    val = gcol * oe

    @pl.when(e == 0)
    def _():
        o_ref[...] = val

    @pl.when(e > 0)
    def _():
        o_ref[...] += val


def _tf_kernel(y_ref, ln1g, ln1b, wqkv, bqkv, wo, bo, ln2g, ln2b,
               wfc1, bfc1, wfc2, bfc2, lnfg, lnfb, wproj, gproj, bproj,
               o_ref, *, S, N, C, Co, heads, hd, depth):
    y = y_ref[...]
    scale = hd ** -0.5
    for d in range(depth):
        h1 = _ln_in(y, ln1g[d], ln1b[d])
        qkv = (jnp.dot(h1.reshape(S * N, C).astype(BF16), wqkv[d],
                       preferred_element_type=F32) + bqkv[d]).reshape(S, N, 3 * C)
        outs = []
        for h in range(heads):
            q = qkv[:, :, h * hd:(h + 1) * hd].astype(BF16)
            k = qkv[:, :, C + h * hd:C + (h + 1) * hd].astype(BF16)
            v = qkv[:, :, 2 * C + h * hd:2 * C + (h + 1) * hd].astype(BF16)
            s = jax.lax.dot_general(q, k, (((2,), (2,)), ((0,), (0,))),
                                    preferred_element_type=F32) * scale
            s = jax.nn.softmax(s, axis=-1)
            o = jax.lax.dot_general(s.astype(BF16), v, (((2,), (1,)), ((0,), (0,))),
                                    preferred_element_type=F32)
            outs.append(o)
        o = jnp.concatenate(outs, axis=-1)
        y = y + (jnp.dot(o.reshape(S * N, C).astype(BF16), wo[d],
                         preferred_element_type=F32) + bo[d]).reshape(S, N, C)
        h2 = _ln_in(y, ln2g[d], ln2b[d])
        f = jnp.dot(h2.reshape(S * N, C).astype(BF16), wfc1[d],
                    preferred_element_type=F32) + bfc1[d]
        f = _silu(f).astype(BF16)
        y = y + (jnp.dot(f, wfc2[d], preferred_element_type=F32)
                 + bfc2[d]).reshape(S, N, C)
    yf = _ln_in(y, lnfg[0], lnfb[0])
    p = jnp.dot(yf.reshape(S * N, C).astype(BF16), wproj[...],
                preferred_element_type=F32)
    p = _silu(p * gproj[...] + bproj[...])
    o_ref[...] = p.reshape(S, N, Co)


def _fus_kernel(xp_ref, yp_ref, wfx_ref, wfy_ref, g_ref, b_ref, o_ref,
                *, rb, W, Cin, Co):
    i = pl.program_id(1)
    xr = xp_ref[0, pl.ds(i * rb, rb + 2)]
    yr = yp_ref[0, pl.ds(i * rb, rb + 2)]
    acc = jnp.zeros((rb * W, Co), F32)
    for dy in range(3):
        for dx in range(3):
            k = 3 * dy + dx
            acc = acc + jnp.dot(xr[dy:dy + rb, dx:dx + W, :].reshape(rb * W, Cin),
                                wfx_ref[k], preferred_element_type=F32)
            acc = acc + jnp.dot(yr[dy:dy + rb, dx:dx + W, :].reshape(rb * W, Co),
                                wfy_ref[k], preferred_element_type=F32)
    o = _silu(acc * g_ref[...] + b_ref[...])
    o_ref[0] = o.reshape(rb, W, Co)


def kernel(x, task_bh, params):
    p = params
    B, Cin, H, W = x.shape
    Cd = p['conv_1x1_w'].shape[0]
    Co = p['conv_proj_w'].shape[0]
    E = p['moe_w1'].shape[0]
    depth = p['wqkv'].shape[0]
    heads = 4
    hd = Cd // heads
    ph = pw = 8
    nph, npw = H // ph, W // pw
    pa, npat = ph * pw, nph * npw
    T = B * H * W

    xcl = jnp.transpose(x, (0, 2, 3, 1))
    xpad = jnp.pad(xcl, ((0, 0), (1, 1), (1, 1), (0, 0)))
    wk = jnp.transpose(p['conv_kxk_w'], (2, 3, 1, 0)).reshape(9, Cin, Cin)
    w1x1 = p['conv_1x1_w'][:, :, 0, 0].T
    gk = p['conv_kxk_g'].reshape(1, Cin)
    bk = p['conv_kxk_b'].reshape(1, Cin)
    RB = 16
    import functools
    ya = pl.pallas_call(
        functools.partial(_stage_a_kernel, rb=RB, W=W, Cin=Cin, Cd=Cd),
        grid=(B, H // RB),
        in_specs=[
            pl.BlockSpec((1, H + 2, W + 2, Cin), lambda b, i: (b, 0, 0, 0)),
            pl.BlockSpec((9, Cin, Cin), lambda b, i: (0, 0, 0)),
            pl.BlockSpec((1, Cin), lambda b, i: (0, 0)),
            pl.BlockSpec((1, Cin), lambda b, i: (0, 0)),
            pl.BlockSpec((Cin, Cd), lambda b, i: (0, 0)),
        ],
        out_specs=pl.BlockSpec((1, RB, W, Cd), lambda b, i: (b, i, 0, 0)),
        out_shape=jax.ShapeDtypeStruct((B, H, W, Cd), F32),
    )(xpad, wk, gk, bk, w1x1)

    yseq = (ya.reshape(B, nph, ph, npw, pw, Cd)
              .transpose(0, 2, 4, 1, 3, 5)
              .reshape(B * pa, npat, Cd))
    xt = yseq.reshape(T, Cd)

    wg = p['w_gate'][task_bh]
    TB_G = 2048
    nblk = T // TB_G
    gates, loss, imp, load = pl.pallas_call(
        functools.partial(_gating_kernel, E=E, nblk=nblk),
        grid=(nblk,),
        in_specs=[
            pl.BlockSpec((TB_G, Cd), lambda t: (t, 0)),
            pl.BlockSpec((Cd, E), lambda t: (0, 0)),
        ],
        out_specs=[
            pl.BlockSpec((TB_G, E), lambda t: (t, 0)),
            pl.BlockSpec((1, 1), lambda t: (0, 0)),
            pl.BlockSpec((1, E), lambda t: (0, 0)),
            pl.BlockSpec((1, E), lambda t: (0, 0)),
        ],
        out_shape=[
            jax.ShapeDtypeStruct((T, E), F32),
            jax.ShapeDtypeStruct((1, 1), F32),
            jax.ShapeDtypeStruct((1, E), F32),
            jax.ShapeDtypeStruct((1, E), F32),
        ],
    )(xt, wg)

    TB_M = 2048
    moe = pl.pallas_call(
        _moe_kernel,
        grid=(T // TB_M, E),
        in_specs=[
            pl.BlockSpec((TB_M, Cd), lambda t, e: (t, 0)),
            pl.BlockSpec((TB_M, E), lambda t, e: (t, 0)),
            pl.BlockSpec((1, Cd, Cd), lambda t, e: (e, 0, 0)),
            pl.BlockSpec((1, 1, Cd), lambda t, e: (e, 0, 0)),
            pl.BlockSpec((1, Cd, Cd), lambda t, e: (e, 0, 0)),
            pl.BlockSpec((1, 1, Cd), lambda t, e: (e, 0, 0)),
        ],
        out_specs=pl.BlockSpec((TB_M, Cd), lambda t, e: (t, 0)),
        out_shape=jax.ShapeDtypeStruct((T, Cd), F32),
    )(xt.astype(BF16), gates, p['moe_w1'].astype(BF16),
      p['moe_b1'].reshape(E, 1, Cd),
      p['moe_w2'].astype(BF16), p['moe_b2'].reshape(E, 1, Cd))

    S = 16
    wproj = p['conv_proj_w'][:, :, 0, 0].T
    ypseq = pl.pallas_call(
        functools.partial(_tf_kernel, S=S, N=npat, C=Cd, Co=Co,
                          heads=heads, hd=hd, depth=depth),
        grid=(B * pa // S,),
        in_specs=[
            pl.BlockSpec((S, npat, Cd), lambda i: (i, 0, 0)),
            pl.BlockSpec((depth, Cd), lambda i: (0, 0)),
            pl.BlockSpec((depth, Cd), lambda i: (0, 0)),
            pl.BlockSpec((depth, Cd, 3 * Cd), lambda i: (0, 0, 0)),
            pl.BlockSpec((depth, 3 * Cd), lambda i: (0, 0)),
            pl.BlockSpec((depth, Cd, Cd), lambda i: (0, 0, 0)),
            pl.BlockSpec((depth, Cd), lambda i: (0, 0)),
            pl.BlockSpec((depth, Cd), lambda i: (0, 0)),
            pl.BlockSpec((depth, Cd), lambda i: (0, 0)),
            pl.BlockSpec((depth, Cd, 2 * Cd), lambda i: (0, 0, 0)),
            pl.BlockSpec((depth, 2 * Cd), lambda i: (0, 0)),
            pl.BlockSpec((depth, 2 * Cd, Cd), lambda i: (0, 0, 0)),
            pl.BlockSpec((depth, Cd), lambda i: (0, 0)),
            pl.BlockSpec((1, Cd), lambda i: (0, 0)),
            pl.BlockSpec((1, Cd), lambda i: (0, 0)),
            pl.BlockSpec((Cd, Co), lambda i: (0, 0)),
            pl.BlockSpec((1, Co), lambda i: (0, 0)),
            pl.BlockSpec((1, Co), lambda i: (0, 0)),
        ],
        out_specs=pl.BlockSpec((S, npat, Co), lambda i: (i, 0, 0)),
        out_shape=jax.ShapeDtypeStruct((B * pa, npat, Co), F32),
    )(moe.reshape(B * pa, npat, Cd),
      p['ln1_g'], p['ln1_b'], p['wqkv'].astype(BF16), p['bqkv'],
      p['wo'].astype(BF16), p['bo'], p['ln2_g'], p['ln2_b'],
      p['wfc1'].astype(BF16), p['bfc1'], p['wfc2'].astype(BF16), p['bfc2'],
      p['lnf_g'].reshape(1, Cd), p['lnf_b'].reshape(1, Cd),
      wproj.astype(BF16), p['conv_proj_g'].reshape(1, Co),
      p['conv_proj_b'].reshape(1, Co))

    yp = (ypseq.reshape(B, ph, pw, nph, npw, Co)
               .transpose(0, 3, 1, 4, 2, 5)
               .reshape(B, H, W, Co))
    yppad = jnp.pad(yp, ((0, 0), (1, 1), (1, 1), (0, 0))).astype(BF16)
    xpad_bf = xpad.astype(BF16)
    wfus = p['conv_fus_w']
    wfx = jnp.transpose(wfus[:, :Cin], (2, 3, 1, 0)).reshape(9, Cin, Co).astype(BF16)
    wfy = jnp.transpose(wfus[:, Cin:], (2, 3, 1, 0)).reshape(9, Co, Co).astype(BF16)
    out = pl.pallas_call(
        functools.partial(_fus_kernel, rb=RB, W=W, Cin=Cin, Co=Co),
        grid=(B, H // RB),
        in_specs=[
            pl.BlockSpec((1, H + 2, W + 2, Cin), lambda b, i: (b, 0, 0, 0)),
            pl.BlockSpec((1, H + 2, W + 2, Co), lambda b, i: (b, 0, 0, 0)),
            pl.BlockSpec((9, Cin, Co), lambda b, i: (0, 0, 0)),
            pl.BlockSpec((9, Co, Co), lambda b, i: (0, 0, 0)),
            pl.BlockSpec((1, Co), lambda b, i: (0, 0)),
            pl.BlockSpec((1, Co), lambda b, i: (0, 0)),
        ],
        out_specs=pl.BlockSpec((1, RB, W, Co), lambda b, i: (b, i, 0, 0)),
        out_shape=jax.ShapeDtypeStruct((B, H, W, Co), F32),
    )(xpad_bf, yppad, wfx, wfy,
      p['conv_fus_g'].reshape(1, Co), p['conv_fus_b'].reshape(1, Co))

    y_final = jnp.transpose(out, (0, 3, 1, 2))
    return y_final, loss.reshape(())


# speedup vs baseline: 3.0603x; 1.1008x over previous
import jax
import jax.numpy as jnp
from jax.experimental import pallas as pl
from jax.experimental.pallas import tpu as pltpu

HIGH = jax.lax.Precision.HIGHEST
F32 = jnp.float32
BF16 = jnp.bfloat16


def _silu(x):
    return x * jax.nn.sigmoid(x)


def _split_hi_lo(v):
    hi = v.astype(BF16)
    lo = (v - hi.astype(F32)).astype(BF16)
    return hi, lo


def _dot3(x, w_hi, w_lo):
    x_hi, x_lo = _split_hi_lo(x)
    acc = jnp.dot(x_hi, w_hi, preferred_element_type=F32)
    acc += jnp.dot(x_hi, w_lo, preferred_element_type=F32)
    acc += jnp.dot(x_lo, w_hi, preferred_element_type=F32)
    return acc


def _ln_in(x, g, b):
    m = jnp.mean(x, -1, keepdims=True)
    v = jnp.mean((x - m) ** 2, -1, keepdims=True)
    return (x - m) * jax.lax.rsqrt(v + 1e-5) * g + b


def _stage_a_kernel(xp_ref, wkh_ref, wkl_ref, g_ref, b_ref, w1h_ref, w1l_ref,
                    o_ref, *, rb, W, Cin, Cd):
    i = pl.program_id(1)
    rows = xp_ref[0, pl.ds(i * rb, rb + 2)]
    rows_hi, rows_lo = _split_hi_lo(rows)
    acc = jnp.zeros((rb * W, Cin), F32)
    for dy in range(3):
        for dx in range(3):
            k = 3 * dy + dx
            xh = rows_hi[dy:dy + rb, dx:dx + W, :].reshape(rb * W, Cin)
            xl = rows_lo[dy:dy + rb, dx:dx + W, :].reshape(rb * W, Cin)
            acc = acc + jnp.dot(xh, wkh_ref[k], preferred_element_type=F32)
            acc = acc + jnp.dot(xh, wkl_ref[k], preferred_element_type=F32)
            acc = acc + jnp.dot(xl, wkh_ref[k], preferred_element_type=F32)
    y = _silu(acc * g_ref[...] + b_ref[...])
    out = _dot3(y, w1h_ref[...], w1l_ref[...])
    o_ref[0] = out.reshape(rb, W, Cd)


def _gating_kernel(x_ref, wgh_ref, wgl_ref, gates_ref, loss_ref, imp_ref,
                   load_ref, *, E, nblk):
    t = pl.program_id(0)
    x = x_ref[...]
    logits = _dot3(x, wgh_ref[...], wgl_ref[...])
    tb = logits.shape[0]
    iota = jax.lax.broadcasted_iota(jnp.int32, (tb, E), 1)
    m1 = jnp.max(logits, axis=1, keepdims=True)
    i1 = jnp.argmax(logits, axis=1)[:, None]
    masked = jnp.where(iota == i1, -jnp.inf, logits)
    m2 = jnp.max(masked, axis=1, keepdims=True)
    i2 = jnp.argmax(masked, axis=1)[:, None]
    w1s = jax.nn.sigmoid(m1 - m2)
    w2s = jax.nn.sigmoid(m2 - m1)
    g = jnp.where(iota == i1, w1s, 0.0) + jnp.where(iota == i2, w2s, 0.0)
    gates_ref[...] = g

    @pl.when(t == 0)
    def _():
        imp_ref[...] = jnp.zeros_like(imp_ref)
        load_ref[...] = jnp.zeros_like(load_ref)

    imp_ref[...] += jnp.sum(g, axis=0, keepdims=True)
    load_ref[...] += jnp.sum((g > 0).astype(F32), axis=0, keepdims=True)

    @pl.when(t == nblk - 1)
    def _():
        def cv2(v):
            m = jnp.mean(v)
            var = jnp.mean((v - m) ** 2)
            return var / (m * m + 1e-10)

        val = cv2(imp_ref[0]) + cv2(load_ref[0])
        loss_ref[...] = jnp.broadcast_to(val, (1, 1))


def _moe_kernel(x_ref, g_ref, w1_ref, b1_ref, w2_ref, b2_ref, o_ref):
    e = pl.program_id(1)
    x = x_ref[...]
    h = jnp.dot(x, w1_ref[0], preferred_element_type=F32) + b1_ref[0]
    h = jnp.maximum(h, 0.0).astype(BF16)
    oe = jnp.dot(h, w2_ref[0], preferred_element_type=F32) + b2_ref[0]
    g8 = g_ref[...]
    iota = jax.lax.broadcasted_iota(jnp.int32, g8.shape, 1)
    gcol = jnp.sum(jnp.where(iota == e, g8, 0.0), axis=1, keepdims=True)
    val = gcol * oe

    @pl.when(e == 0)
    def _():
        o_ref[...] = val

    @pl.when(e > 0)
    def _():
        o_ref[...] += val


def _tf_kernel(y_ref, ln1g, ln1b, wqkv, bqkv, wo, bo, ln2g, ln2b,
               wfc1, bfc1, wfc2, bfc2, lnfg, lnfb, wproj, gproj, bproj,
               o_ref, *, S, N, C, Co, heads, hd, depth):
    y = y_ref[...]
    scale = hd ** -0.5
    for d in range(depth):
        h1 = _ln_in(y, ln1g[d], ln1b[d])
        qkv = (jnp.dot(h1.reshape(S * N, C).astype(BF16), wqkv[d],
                       preferred_element_type=F32) + bqkv[d]).reshape(S, N, 3 * C)
        outs = []
        for h in range(heads):
            q = qkv[:, :, h * hd:(h + 1) * hd].astype(BF16)
            k = qkv[:, :, C + h * hd:C + (h + 1) * hd].astype(BF16)
            v = qkv[:, :, 2 * C + h * hd:2 * C + (h + 1) * hd].astype(BF16)
            s = jax.lax.dot_general(q, k, (((2,), (2,)), ((0,), (0,))),
                                    preferred_element_type=F32) * scale
            s = jax.nn.softmax(s, axis=-1)
            o = jax.lax.dot_general(s.astype(BF16), v, (((2,), (1,)), ((0,), (0,))),
                                    preferred_element_type=F32)
            outs.append(o)
        o = jnp.concatenate(outs, axis=-1)
        y = y + (jnp.dot(o.reshape(S * N, C).astype(BF16), wo[d],
                         preferred_element_type=F32) + bo[d]).reshape(S, N, C)
        h2 = _ln_in(y, ln2g[d], ln2b[d])
        f = jnp.dot(h2.reshape(S * N, C).astype(BF16), wfc1[d],
                    preferred_element_type=F32) + bfc1[d]
        f = _silu(f).astype(BF16)
        y = y + (jnp.dot(f, wfc2[d], preferred_element_type=F32)
                 + bfc2[d]).reshape(S, N, C)
    yf = _ln_in(y, lnfg[0], lnfb[0])
    p = jnp.dot(yf.reshape(S * N, C).astype(BF16), wproj[...],
                preferred_element_type=F32)
    p = _silu(p * gproj[...] + bproj[...])
    o_ref[...] = p.reshape(S, N, Co)


def _fus_kernel(xp_ref, yp_ref, wfx_ref, wfy_ref, g_ref, b_ref, o_ref,
                *, rb, W, Cin, Co):
    i = pl.program_id(1)
    xr = xp_ref[0, pl.ds(i * rb, rb + 2)]
    yr = yp_ref[0, pl.ds(i * rb, rb + 2)]
    acc = jnp.zeros((rb * W, Co), F32)
    for dy in range(3):
        for dx in range(3):
            k = 3 * dy + dx
            acc = acc + jnp.dot(xr[dy:dy + rb, dx:dx + W, :].reshape(rb * W, Cin),
                                wfx_ref[k], preferred_element_type=F32)
            acc = acc + jnp.dot(yr[dy:dy + rb, dx:dx + W, :].reshape(rb * W, Co),
                                wfy_ref[k], preferred_element_type=F32)
    o = _silu(acc * g_ref[...] + b_ref[...])
    o_ref[0] = o.reshape(rb, W, Co)


def kernel(x, task_bh, params):
    p = params
    B, Cin, H, W = x.shape
    Cd = p['conv_1x1_w'].shape[0]
    Co = p['conv_proj_w'].shape[0]
    E = p['moe_w1'].shape[0]
    depth = p['wqkv'].shape[0]
    heads = 4
    hd = Cd // heads
    ph = pw = 8
    nph, npw = H // ph, W // pw
    pa, npat = ph * pw, nph * npw
    T = B * H * W

    xcl = jnp.transpose(x, (0, 2, 3, 1))
    xpad = jnp.pad(xcl, ((0, 0), (1, 1), (1, 1), (0, 0)))
    wk = jnp.transpose(p['conv_kxk_w'], (2, 3, 1, 0)).reshape(9, Cin, Cin)
    w1x1 = p['conv_1x1_w'][:, :, 0, 0].T
    wk_hi = wk.astype(BF16)
    wk_lo = (wk - wk_hi.astype(F32)).astype(BF16)
    w1_hi = w1x1.astype(BF16)
    w1_lo = (w1x1 - w1_hi.astype(F32)).astype(BF16)
    gk = p['conv_kxk_g'].reshape(1, Cin)
    bk = p['conv_kxk_b'].reshape(1, Cin)
    RB = 16
    import functools
    ya = pl.pallas_call(
        functools.partial(_stage_a_kernel, rb=RB, W=W, Cin=Cin, Cd=Cd),
        grid=(B, H // RB),
        in_specs=[
            pl.BlockSpec((1, H + 2, W + 2, Cin), lambda b, i: (b, 0, 0, 0)),
            pl.BlockSpec((9, Cin, Cin), lambda b, i: (0, 0, 0)),
            pl.BlockSpec((9, Cin, Cin), lambda b, i: (0, 0, 0)),
            pl.BlockSpec((1, Cin), lambda b, i: (0, 0)),
            pl.BlockSpec((1, Cin), lambda b, i: (0, 0)),
            pl.BlockSpec((Cin, Cd), lambda b, i: (0, 0)),
            pl.BlockSpec((Cin, Cd), lambda b, i: (0, 0)),
        ],
        out_specs=pl.BlockSpec((1, RB, W, Cd), lambda b, i: (b, i, 0, 0)),
        out_shape=jax.ShapeDtypeStruct((B, H, W, Cd), F32),
    )(xpad, wk_hi, wk_lo, gk, bk, w1_hi, w1_lo)

    yseq = (ya.reshape(B, nph, ph, npw, pw, Cd)
              .transpose(0, 2, 4, 1, 3, 5)
              .reshape(B * pa, npat, Cd))
    xt = yseq.reshape(T, Cd)

    wg = p['w_gate'][task_bh]
    wg_hi = wg.astype(BF16)
    wg_lo = (wg - wg_hi.astype(F32)).astype(BF16)
    TB_G = 2048
    nblk = T // TB_G
    gates, loss, imp, load = pl.pallas_call(
        functools.partial(_gating_kernel, E=E, nblk=nblk),
        grid=(nblk,),
        in_specs=[
            pl.BlockSpec((TB_G, Cd), lambda t: (t, 0)),
            pl.BlockSpec((Cd, E), lambda t: (0, 0)),
            pl.BlockSpec((Cd, E), lambda t: (0, 0)),
        ],
        out_specs=[
            pl.BlockSpec((TB_G, E), lambda t: (t, 0)),
            pl.BlockSpec((1, 1), lambda t: (0, 0)),
            pl.BlockSpec((1, E), lambda t: (0, 0)),
            pl.BlockSpec((1, E), lambda t: (0, 0)),
        ],
        out_shape=[
            jax.ShapeDtypeStruct((T, E), F32),
            jax.ShapeDtypeStruct((1, 1), F32),
            jax.ShapeDtypeStruct((1, E), F32),
            jax.ShapeDtypeStruct((1, E), F32),
        ],
    )(xt, wg_hi, wg_lo)

    TB_M = 2048
    moe = pl.pallas_call(
        _moe_kernel,
        grid=(T // TB_M, E),
        in_specs=[
            pl.BlockSpec((TB_M, Cd), lambda t, e: (t, 0)),
            pl.BlockSpec((TB_M, E), lambda t, e: (t, 0)),
            pl.BlockSpec((1, Cd, Cd), lambda t, e: (e, 0, 0)),
            pl.BlockSpec((1, 1, Cd), lambda t, e: (e, 0, 0)),
            pl.BlockSpec((1, Cd, Cd), lambda t, e: (e, 0, 0)),
            pl.BlockSpec((1, 1, Cd), lambda t, e: (e, 0, 0)),
        ],
        out_specs=pl.BlockSpec((TB_M, Cd), lambda t, e: (t, 0)),
        out_shape=jax.ShapeDtypeStruct((T, Cd), F32),
    )(xt.astype(BF16), gates, p['moe_w1'].astype(BF16),
      p['moe_b1'].reshape(E, 1, Cd),
      p['moe_w2'].astype(BF16), p['moe_b2'].reshape(E, 1, Cd))

    S = 32
    wproj = p['conv_proj_w'][:, :, 0, 0].T
    ypseq = pl.pallas_call(
        functools.partial(_tf_kernel, S=S, N=npat, C=Cd, Co=Co,
                          heads=heads, hd=hd, depth=depth),
        grid=(B * pa // S,),
        in_specs=[
            pl.BlockSpec((S, npat, Cd), lambda i: (i, 0, 0)),
            pl.BlockSpec((depth, Cd), lambda i: (0, 0)),
            pl.BlockSpec((depth, Cd), lambda i: (0, 0)),
            pl.BlockSpec((depth, Cd, 3 * Cd), lambda i: (0, 0, 0)),
            pl.BlockSpec((depth, 3 * Cd), lambda i: (0, 0)),
            pl.BlockSpec((depth, Cd, Cd), lambda i: (0, 0, 0)),
            pl.BlockSpec((depth, Cd), lambda i: (0, 0)),
            pl.BlockSpec((depth, Cd), lambda i: (0, 0)),
            pl.BlockSpec((depth, Cd), lambda i: (0, 0)),
            pl.BlockSpec((depth, Cd, 2 * Cd), lambda i: (0, 0, 0)),
            pl.BlockSpec((depth, 2 * Cd), lambda i: (0, 0)),
            pl.BlockSpec((depth, 2 * Cd, Cd), lambda i: (0, 0, 0)),
            pl.BlockSpec((depth, Cd), lambda i: (0, 0)),
            pl.BlockSpec((1, Cd), lambda i: (0, 0)),
            pl.BlockSpec((1, Cd), lambda i: (0, 0)),
            pl.BlockSpec((Cd, Co), lambda i: (0, 0)),
            pl.BlockSpec((1, Co), lambda i: (0, 0)),
            pl.BlockSpec((1, Co), lambda i: (0, 0)),
        ],
        out_specs=pl.BlockSpec((S, npat, Co), lambda i: (i, 0, 0)),
        out_shape=jax.ShapeDtypeStruct((B * pa, npat, Co), F32),
    )(moe.reshape(B * pa, npat, Cd),
      p['ln1_g'], p['ln1_b'], p['wqkv'].astype(BF16), p['bqkv'],
      p['wo'].astype(BF16), p['bo'], p['ln2_g'], p['ln2_b'],
      p['wfc1'].astype(BF16), p['bfc1'], p['wfc2'].astype(BF16), p['bfc2'],
      p['lnf_g'].reshape(1, Cd), p['lnf_b'].reshape(1, Cd),
      wproj.astype(BF16), p['conv_proj_g'].reshape(1, Co),
      p['conv_proj_b'].reshape(1, Co))

    yp = (ypseq.reshape(B, ph, pw, nph, npw, Co)
               .transpose(0, 3, 1, 4, 2, 5)
               .reshape(B, H, W, Co))
    yppad = jnp.pad(yp, ((0, 0), (1, 1), (1, 1), (0, 0))).astype(BF16)
    xpad_bf = xpad.astype(BF16)
    wfus = p['conv_fus_w']
    wfx = jnp.transpose(wfus[:, :Cin], (2, 3, 1, 0)).reshape(9, Cin, Co).astype(BF16)
    wfy = jnp.transpose(wfus[:, Cin:], (2, 3, 1, 0)).reshape(9, Co, Co).astype(BF16)
    out = pl.pallas_call(
        functools.partial(_fus_kernel, rb=RB, W=W, Cin=Cin, Co=Co),
        grid=(B, H // RB),
        in_specs=[
            pl.BlockSpec((1, H + 2, W + 2, Cin), lambda b, i: (b, 0, 0, 0)),
            pl.BlockSpec((1, H + 2, W + 2, Co), lambda b, i: (b, 0, 0, 0)),
            pl.BlockSpec((9, Cin, Co), lambda b, i: (0, 0, 0)),
            pl.BlockSpec((9, Co, Co), lambda b, i: (0, 0, 0)),
            pl.BlockSpec((1, Co), lambda b, i: (0, 0)),
            pl.BlockSpec((1, Co), lambda b, i: (0, 0)),
        ],
        out_specs=pl.BlockSpec((1, RB, W, Co), lambda b, i: (b, i, 0, 0)),
        out_shape=jax.ShapeDtypeStruct((B, H, W, Co), F32),
    )(xpad_bf, yppad, wfx, wfy,
      p['conv_fus_g'].reshape(1, Co), p['conv_fus_b'].reshape(1, Co))

    y_final = jnp.transpose(out, (0, 3, 1, 2))
    return y_final, loss.reshape(())


# speedup vs baseline: 3.1586x; 1.0321x over previous
import jax
import jax.numpy as jnp
from jax.experimental import pallas as pl
from jax.experimental.pallas import tpu as pltpu

HIGH = jax.lax.Precision.HIGHEST
F32 = jnp.float32
BF16 = jnp.bfloat16


def _silu(x):
    return x * jax.nn.sigmoid(x)


def _split_hi_lo(v):
    hi = v.astype(BF16)
    lo = (v - hi.astype(F32)).astype(BF16)
    return hi, lo


def _dot3(x, w_hi, w_lo):
    x_hi, x_lo = _split_hi_lo(x)
    acc = jnp.dot(x_hi, w_hi, preferred_element_type=F32)
    acc += jnp.dot(x_hi, w_lo, preferred_element_type=F32)
    acc += jnp.dot(x_lo, w_hi, preferred_element_type=F32)
    return acc


def _ln_in(x, g, b):
    m = jnp.mean(x, -1, keepdims=True)
    v = jnp.mean((x - m) ** 2, -1, keepdims=True)
    return (x - m) * jax.lax.rsqrt(v + 1e-5) * g + b


def _stage_a_kernel(xp_ref, wkh_ref, wkl_ref, g_ref, b_ref, w1h_ref, w1l_ref,
                    o_ref, *, rb, W, Cin, Cd):
    i = pl.program_id(1)
    rows = xp_ref[0, pl.ds(i * rb, rb + 2)]
    rows_hi, rows_lo = _split_hi_lo(rows)
    acc = jnp.zeros((rb * W, Cin), F32)
    for dy in range(3):
        for dx in range(3):
            k = 3 * dy + dx
            xh = rows_hi[dy:dy + rb, dx:dx + W, :].reshape(rb * W, Cin)
            xl = rows_lo[dy:dy + rb, dx:dx + W, :].reshape(rb * W, Cin)
            acc = acc + jnp.dot(xh, wkh_ref[k], preferred_element_type=F32)
            acc = acc + jnp.dot(xh, wkl_ref[k], preferred_element_type=F32)
            acc = acc + jnp.dot(xl, wkh_ref[k], preferred_element_type=F32)
    y = _silu(acc * g_ref[...] + b_ref[...])
    out = _dot3(y, w1h_ref[...], w1l_ref[...])
    o_ref[0] = out.reshape(rb, W, Cd)


def _gating_kernel(x_ref, wgh_ref, wgl_ref, gates_ref, loss_ref, imp_ref,
                   load_ref, *, E, nblk):
    t = pl.program_id(0)
    x = x_ref[...]
    logits = _dot3(x, wgh_ref[...], wgl_ref[...])
    tb = logits.shape[0]
    iota = jax.lax.broadcasted_iota(jnp.int32, (tb, E), 1)
    m1 = jnp.max(logits, axis=1, keepdims=True)
    i1 = jnp.argmax(logits, axis=1)[:, None]
    masked = jnp.where(iota == i1, -jnp.inf, logits)
    m2 = jnp.max(masked, axis=1, keepdims=True)
    i2 = jnp.argmax(masked, axis=1)[:, None]
    w1s = jax.nn.sigmoid(m1 - m2)
    w2s = jax.nn.sigmoid(m2 - m1)
    g = jnp.where(iota == i1, w1s, 0.0) + jnp.where(iota == i2, w2s, 0.0)
    gates_ref[...] = g

    @pl.when(t == 0)
    def _():
        imp_ref[...] = jnp.zeros_like(imp_ref)
        load_ref[...] = jnp.zeros_like(load_ref)

    imp_ref[...] += jnp.sum(g, axis=0, keepdims=True)
    load_ref[...] += jnp.sum((g > 0).astype(F32), axis=0, keepdims=True)

    @pl.when(t == nblk - 1)
    def _():
        def cv2(v):
            m = jnp.mean(v)
            var = jnp.mean((v - m) ** 2)
            return var / (m * m + 1e-10)

        val = cv2(imp_ref[0]) + cv2(load_ref[0])
        loss_ref[...] = jnp.broadcast_to(val, (1, 1))


def _moe_kernel(x_ref, g_ref, w1_ref, b1_ref, w2_ref, b2_ref, o_ref, *, Cd):
    e = pl.program_id(1)
    x = x_ref[...]
    tb = x.shape[0]
    h = jnp.dot(x, w1_ref[0], preferred_element_type=F32) + b1_ref[0]
    h = jnp.maximum(h, 0.0)
    g8 = g_ref[...]
    iota = jax.lax.broadcasted_iota(jnp.int32, g8.shape, 1)
    ga = jnp.sum(jnp.where(iota == 2 * e, g8, 0.0), axis=1, keepdims=True)
    gb = jnp.sum(jnp.where(iota == 2 * e + 1, g8, 0.0), axis=1, keepdims=True)
    gh = jnp.concatenate(
        [jnp.broadcast_to(ga, (tb, Cd)), jnp.broadcast_to(gb, (tb, Cd))], axis=1)
    hg = (h * gh).astype(BF16)
    oe = jnp.dot(hg, w2_ref[0], preferred_element_type=F32)
    oe += ga * b2_ref[0, :, :Cd] + gb * b2_ref[0, :, Cd:]
    @pl.when(e == 0)
    def _():
        o_ref[...] = oe

    @pl.when(e > 0)
    def _():
        o_ref[...] += oe


def _tf_kernel(y_ref, ln1g, ln1b, wqkv, bqkv, wo, bo, ln2g, ln2b,
               wfc1, bfc1, wfc2, bfc2, lnfg, lnfb, wproj, gproj, bproj,
               o_ref, *, S, N, C, Co, heads, hd, depth):
    y = y_ref[...]
    scale = hd ** -0.5
    for d in range(depth):
        h1 = _ln_in(y, ln1g[d], ln1b[d])
        qkv = (jnp.dot(h1.reshape(S * N, C).astype(BF16), wqkv[d],
                       preferred_element_type=F32) + bqkv[d]).reshape(S, N, 3 * C)
        outs = []
        for h in range(heads):
            q = qkv[:, :, h * hd:(h + 1) * hd].astype(BF16)
            k = qkv[:, :, C + h * hd:C + (h + 1) * hd].astype(BF16)
            v = qkv[:, :, 2 * C + h * hd:2 * C + (h + 1) * hd].astype(BF16)
            s = jax.lax.dot_general(q, k, (((2,), (2,)), ((0,), (0,))),
                                    preferred_element_type=F32) * scale
            s = jax.nn.softmax(s, axis=-1)
            o = jax.lax.dot_general(s.astype(BF16), v, (((2,), (1,)), ((0,), (0,))),
                                    preferred_element_type=F32)
            outs.append(o)
        o = jnp.concatenate(outs, axis=-1)
        y = y + (jnp.dot(o.reshape(S * N, C).astype(BF16), wo[d],
                         preferred_element_type=F32) + bo[d]).reshape(S, N, C)
        h2 = _ln_in(y, ln2g[d], ln2b[d])
        f = jnp.dot(h2.reshape(S * N, C).astype(BF16), wfc1[d],
                    preferred_element_type=F32) + bfc1[d]
        f = _silu(f).astype(BF16)
        y = y + (jnp.dot(f, wfc2[d], preferred_element_type=F32)
                 + bfc2[d]).reshape(S, N, C)
    yf = _ln_in(y, lnfg[0], lnfb[0])
    p = jnp.dot(yf.reshape(S * N, C).astype(BF16), wproj[...],
                preferred_element_type=F32)
    p = _silu(p * gproj[...] + bproj[...])
    o_ref[...] = p.reshape(S, N, Co)


def _fus_kernel(xp_ref, yp_ref, wfx_ref, wfy_ref, g_ref, b_ref, o_ref,
                *, rb, W, Cin, Co):
    i = pl.program_id(1)
    xr = xp_ref[0, pl.ds(i * rb, rb + 2)]
    yr = yp_ref[0, pl.ds(i * rb, rb + 2)]
    acc = jnp.zeros((rb * W, Co), F32)
    for dy in range(3):
        for dx in range(3):
            k = 3 * dy + dx
            acc = acc + jnp.dot(xr[dy:dy + rb, dx:dx + W, :].reshape(rb * W, Cin),
                                wfx_ref[k], preferred_element_type=F32)
            acc = acc + jnp.dot(yr[dy:dy + rb, dx:dx + W, :].reshape(rb * W, Co),
                                wfy_ref[k], preferred_element_type=F32)
    o = _silu(acc * g_ref[...] + b_ref[...])
    o_ref[0] = o.reshape(rb, W, Co)


def kernel(x, task_bh, params):
    p = params
    B, Cin, H, W = x.shape
    Cd = p['conv_1x1_w'].shape[0]
    Co = p['conv_proj_w'].shape[0]
    E = p['moe_w1'].shape[0]
    depth = p['wqkv'].shape[0]
    heads = 4
    hd = Cd // heads
    ph = pw = 8
    nph, npw = H // ph, W // pw
    pa, npat = ph * pw, nph * npw
    T = B * H * W

    xcl = jnp.transpose(x, (0, 2, 3, 1))
    xpad = jnp.pad(xcl, ((0, 0), (1, 1), (1, 1), (0, 0)))
    wk = jnp.transpose(p['conv_kxk_w'], (2, 3, 1, 0)).reshape(9, Cin, Cin)
    w1x1 = p['conv_1x1_w'][:, :, 0, 0].T
    wk_hi = wk.astype(BF16)
    wk_lo = (wk - wk_hi.astype(F32)).astype(BF16)
    w1_hi = w1x1.astype(BF16)
    w1_lo = (w1x1 - w1_hi.astype(F32)).astype(BF16)
    gk = p['conv_kxk_g'].reshape(1, Cin)
    bk = p['conv_kxk_b'].reshape(1, Cin)
    RB = 16
    import functools
    ya = pl.pallas_call(
        functools.partial(_stage_a_kernel, rb=RB, W=W, Cin=Cin, Cd=Cd),
        grid=(B, H // RB),
        in_specs=[
            pl.BlockSpec((1, H + 2, W + 2, Cin), lambda b, i: (b, 0, 0, 0)),
            pl.BlockSpec((9, Cin, Cin), lambda b, i: (0, 0, 0)),
            pl.BlockSpec((9, Cin, Cin), lambda b, i: (0, 0, 0)),
            pl.BlockSpec((1, Cin), lambda b, i: (0, 0)),
            pl.BlockSpec((1, Cin), lambda b, i: (0, 0)),
            pl.BlockSpec((Cin, Cd), lambda b, i: (0, 0)),
            pl.BlockSpec((Cin, Cd), lambda b, i: (0, 0)),
        ],
        out_specs=pl.BlockSpec((1, RB, W, Cd), lambda b, i: (b, i, 0, 0)),
        out_shape=jax.ShapeDtypeStruct((B, H, W, Cd), F32),
    )(xpad, wk_hi, wk_lo, gk, bk, w1_hi, w1_lo)

    yseq = (ya.reshape(B, nph, ph, npw, pw, Cd)
              .transpose(0, 2, 4, 1, 3, 5)
              .reshape(B * pa, npat, Cd))
    xt = yseq.reshape(T, Cd)

    wg = p['w_gate'][task_bh]
    wg_hi = wg.astype(BF16)
    wg_lo = (wg - wg_hi.astype(F32)).astype(BF16)
    TB_G = 2048
    nblk = T // TB_G
    gates, loss, imp, load = pl.pallas_call(
        functools.partial(_gating_kernel, E=E, nblk=nblk),
        grid=(nblk,),
        in_specs=[
            pl.BlockSpec((TB_G, Cd), lambda t: (t, 0)),
            pl.BlockSpec((Cd, E), lambda t: (0, 0)),
            pl.BlockSpec((Cd, E), lambda t: (0, 0)),
        ],
        out_specs=[
            pl.BlockSpec((TB_G, E), lambda t: (t, 0)),
            pl.BlockSpec((1, 1), lambda t: (0, 0)),
            pl.BlockSpec((1, E), lambda t: (0, 0)),
            pl.BlockSpec((1, E), lambda t: (0, 0)),
        ],
        out_shape=[
            jax.ShapeDtypeStruct((T, E), F32),
            jax.ShapeDtypeStruct((1, 1), F32),
            jax.ShapeDtypeStruct((1, E), F32),
            jax.ShapeDtypeStruct((1, E), F32),
        ],
    )(xt, wg_hi, wg_lo)

    TB_M = 2048
    EP = E // 2
    w1p = (p['moe_w1'].reshape(EP, 2, Cd, Cd).transpose(0, 2, 1, 3)
           .reshape(EP, Cd, 2 * Cd)).astype(BF16)
    b1p = p['moe_b1'].reshape(EP, 1, 2 * Cd)
    w2p = p['moe_w2'].reshape(EP, 2 * Cd, Cd).astype(BF16)
    b2p = p['moe_b2'].reshape(EP, 1, 2 * Cd)
    moe = pl.pallas_call(
        functools.partial(_moe_kernel, Cd=Cd),
        grid=(T // TB_M, EP),
        in_specs=[
            pl.BlockSpec((TB_M, Cd), lambda t, e: (t, 0)),
            pl.BlockSpec((TB_M, E), lambda t, e: (t, 0)),
            pl.BlockSpec((1, Cd, 2 * Cd), lambda t, e: (e, 0, 0)),
            pl.BlockSpec((1, 1, 2 * Cd), lambda t, e: (e, 0, 0)),
            pl.BlockSpec((1, 2 * Cd, Cd), lambda t, e: (e, 0, 0)),
            pl.BlockSpec((1, 1, 2 * Cd), lambda t, e: (e, 0, 0)),
        ],
        out_specs=pl.BlockSpec((TB_M, Cd), lambda t, e: (t, 0)),
        out_shape=jax.ShapeDtypeStruct((T, Cd), F32),
    )(xt.astype(BF16), gates, w1p, b1p, w2p, b2p)

    S = 32
    wproj = p['conv_proj_w'][:, :, 0, 0].T
    ypseq = pl.pallas_call(
        functools.partial(_tf_kernel, S=S, N=npat, C=Cd, Co=Co,
                          heads=heads, hd=hd, depth=depth),
        grid=(B * pa // S,),
        in_specs=[
            pl.BlockSpec((S, npat, Cd), lambda i: (i, 0, 0)),
            pl.BlockSpec((depth, Cd), lambda i: (0, 0)),
            pl.BlockSpec((depth, Cd), lambda i: (0, 0)),
            pl.BlockSpec((depth, Cd, 3 * Cd), lambda i: (0, 0, 0)),
            pl.BlockSpec((depth, 3 * Cd), lambda i: (0, 0)),
            pl.BlockSpec((depth, Cd, Cd), lambda i: (0, 0, 0)),
            pl.BlockSpec((depth, Cd), lambda i: (0, 0)),
            pl.BlockSpec((depth, Cd), lambda i: (0, 0)),
            pl.BlockSpec((depth, Cd), lambda i: (0, 0)),
            pl.BlockSpec((depth, Cd, 2 * Cd), lambda i: (0, 0, 0)),
            pl.BlockSpec((depth, 2 * Cd), lambda i: (0, 0)),
            pl.BlockSpec((depth, 2 * Cd, Cd), lambda i: (0, 0, 0)),
            pl.BlockSpec((depth, Cd), lambda i: (0, 0)),
            pl.BlockSpec((1, Cd), lambda i: (0, 0)),
            pl.BlockSpec((1, Cd), lambda i: (0, 0)),
            pl.BlockSpec((Cd, Co), lambda i: (0, 0)),
            pl.BlockSpec((1, Co), lambda i: (0, 0)),
            pl.BlockSpec((1, Co), lambda i: (0, 0)),
        ],
        out_specs=pl.BlockSpec((S, npat, Co), lambda i: (i, 0, 0)),
        out_shape=jax.ShapeDtypeStruct((B * pa, npat, Co), F32),
    )(moe.reshape(B * pa, npat, Cd),
      p['ln1_g'], p['ln1_b'], p['wqkv'].astype(BF16), p['bqkv'],
      p['wo'].astype(BF16), p['bo'], p['ln2_g'], p['ln2_b'],
      p['wfc1'].astype(BF16), p['bfc1'], p['wfc2'].astype(BF16), p['bfc2'],
      p['lnf_g'].reshape(1, Cd), p['lnf_b'].reshape(1, Cd),
      wproj.astype(BF16), p['conv_proj_g'].reshape(1, Co),
      p['conv_proj_b'].reshape(1, Co))

    yp = (ypseq.reshape(B, ph, pw, nph, npw, Co)
               .transpose(0, 3, 1, 4, 2, 5)
               .reshape(B, H, W, Co))
    yppad = jnp.pad(yp, ((0, 0), (1, 1), (1, 1), (0, 0))).astype(BF16)
    xpad_bf = xpad.astype(BF16)
    wfus = p['conv_fus_w']
    wfx = jnp.transpose(wfus[:, :Cin], (2, 3, 1, 0)).reshape(9, Cin, Co).astype(BF16)
    wfy = jnp.transpose(wfus[:, Cin:], (2, 3, 1, 0)).reshape(9, Co, Co).astype(BF16)
    out = pl.pallas_call(
        functools.partial(_fus_kernel, rb=RB, W=W, Cin=Cin, Co=Co),
        grid=(B, H // RB),
        in_specs=[
            pl.BlockSpec((1, H + 2, W + 2, Cin), lambda b, i: (b, 0, 0, 0)),
            pl.BlockSpec((1, H + 2, W + 2, Co), lambda b, i: (b, 0, 0, 0)),
            pl.BlockSpec((9, Cin, Co), lambda b, i: (0, 0, 0)),
            pl.BlockSpec((9, Co, Co), lambda b, i: (0, 0, 0)),
            pl.BlockSpec((1, Co), lambda b, i: (0, 0)),
            pl.BlockSpec((1, Co), lambda b, i: (0, 0)),
        ],
        out_specs=pl.BlockSpec((1, RB, W, Co), lambda b, i: (b, i, 0, 0)),
        out_shape=jax.ShapeDtypeStruct((B, H, W, Co), F32),
    )(xpad_bf, yppad, wfx, wfy,
      p['conv_fus_g'].reshape(1, Co), p['conv_fus_b'].reshape(1, Co))

    y_final = jnp.transpose(out, (0, 3, 1, 2))
    return y_final, loss.reshape(())


# speedup vs baseline: 3.6510x; 1.1559x over previous
import jax
import jax.numpy as jnp
from jax.experimental import pallas as pl
from jax.experimental.pallas import tpu as pltpu

HIGH = jax.lax.Precision.HIGHEST
F32 = jnp.float32
BF16 = jnp.bfloat16


def _silu(x):
    return x * jax.nn.sigmoid(x)


def _split_hi_lo(v):
    hi = v.astype(BF16)
    lo = (v - hi.astype(F32)).astype(BF16)
    return hi, lo


def _dot3(x, w_hi, w_lo):
    x_hi, x_lo = _split_hi_lo(x)
    acc = jnp.dot(x_hi, w_hi, preferred_element_type=F32)
    acc += jnp.dot(x_hi, w_lo, preferred_element_type=F32)
    acc += jnp.dot(x_lo, w_hi, preferred_element_type=F32)
    return acc


def _ln_in(x, g, b):
    m = jnp.mean(x, -1, keepdims=True)
    v = jnp.mean((x - m) ** 2, -1, keepdims=True)
    return (x - m) * jax.lax.rsqrt(v + 1e-5) * g + b


def _stage_a_kernel(xp_ref, wkh_ref, wkl_ref, g_ref, b_ref, w1h_ref, w1l_ref,
                    o_ref, *, rb, W, Cin, Cd):
    i = pl.program_id(1)
    rows = xp_ref[0, pl.ds(i * rb, rb + 2)]
    rows_hi, rows_lo = _split_hi_lo(rows)
    acc = jnp.zeros((rb * W, Cin), F32)
    for dy in range(3):
        for dx in range(3):
            k = 3 * dy + dx
            xh = rows_hi[dy:dy + rb, dx:dx + W, :].reshape(rb * W, Cin)
            xl = rows_lo[dy:dy + rb, dx:dx + W, :].reshape(rb * W, Cin)
            acc = acc + jnp.dot(xh, wkh_ref[k], preferred_element_type=F32)
            acc = acc + jnp.dot(xh, wkl_ref[k], preferred_element_type=F32)
            acc = acc + jnp.dot(xl, wkh_ref[k], preferred_element_type=F32)
    y = _silu(acc * g_ref[...] + b_ref[...])
    out = _dot3(y, w1h_ref[...], w1l_ref[...])
    o_ref[0] = out.reshape(rb, W, Cd)


def _gating_kernel(x_ref, wgh_ref, wgl_ref, gates_ref, loss_ref, imp_ref,
                   load_ref, *, E, nblk):
    t = pl.program_id(0)
    x = x_ref[...]
    logits = _dot3(x, wgh_ref[...], wgl_ref[...])
    tb = logits.shape[0]
    iota = jax.lax.broadcasted_iota(jnp.int32, (tb, E), 1)
    m1 = jnp.max(logits, axis=1, keepdims=True)
    i1 = jnp.argmax(logits, axis=1)[:, None]
    masked = jnp.where(iota == i1, -jnp.inf, logits)
    m2 = jnp.max(masked, axis=1, keepdims=True)
    i2 = jnp.argmax(masked, axis=1)[:, None]
    w1s = jax.nn.sigmoid(m1 - m2)
    w2s = jax.nn.sigmoid(m2 - m1)
    g = jnp.where(iota == i1, w1s, 0.0) + jnp.where(iota == i2, w2s, 0.0)
    gates_ref[...] = g

    @pl.when(t == 0)
    def _():
        imp_ref[...] = jnp.zeros_like(imp_ref)
        load_ref[...] = jnp.zeros_like(load_ref)

    imp_ref[...] += jnp.sum(g, axis=0, keepdims=True)
    load_ref[...] += jnp.sum((g > 0).astype(F32), axis=0, keepdims=True)

    @pl.when(t == nblk - 1)
    def _():
        def cv2(v):
            m = jnp.mean(v)
            var = jnp.mean((v - m) ** 2)
            return var / (m * m + 1e-10)

        val = cv2(imp_ref[0]) + cv2(load_ref[0])
        loss_ref[...] = jnp.broadcast_to(val, (1, 1))


def _moe_kernel(x_ref, g_ref, w1_ref, b1_ref, w2_ref, b2_ref, o_ref, *, Cd):
    e = pl.program_id(1)
    x = x_ref[...]
    tb = x.shape[0]
    h = jnp.dot(x, w1_ref[0], preferred_element_type=F32) + b1_ref[0]
    h = jnp.maximum(h, 0.0)
    g8 = g_ref[...]
    iota = jax.lax.broadcasted_iota(jnp.int32, g8.shape, 1)
    ga = jnp.sum(jnp.where(iota == 2 * e, g8, 0.0), axis=1, keepdims=True)
    gb = jnp.sum(jnp.where(iota == 2 * e + 1, g8, 0.0), axis=1, keepdims=True)
    gh = jnp.concatenate(
        [jnp.broadcast_to(ga, (tb, Cd)), jnp.broadcast_to(gb, (tb, Cd))], axis=1)
    hg = (h * gh).astype(BF16)
    oe = jnp.dot(hg, w2_ref[0], preferred_element_type=F32)
    oe += ga * b2_ref[0, :, :Cd] + gb * b2_ref[0, :, Cd:]
    @pl.when(e == 0)
    def _():
        o_ref[...] = oe

    @pl.when(e > 0)
    def _():
        o_ref[...] += oe


def _tf_kernel(y_ref, ln1g, ln1b, wqkv, bqkv, wo, bo, ln2g, ln2b,
               wfc1, bfc1, wfc2, bfc2, lnfg, lnfb, wproj, gproj, bproj,
               o_ref, *, S, N, C, Co, heads, hd, depth):
    y = y_ref[...]
    scale = hd ** -0.5
    hp = 128
    for d in range(depth):
        h1 = _ln_in(y, ln1g[d], ln1b[d])
        qkv = (jnp.dot(h1.reshape(S * N, C).astype(BF16), wqkv[d],
                       preferred_element_type=F32)
               + bqkv[d]).reshape(S, N, 3 * heads * hp)
        outs = []
        for h in range(heads):
            off = h * 3 * hp
            q = qkv[:, :, off:off + hp].astype(BF16)
            k = qkv[:, :, off + hp:off + 2 * hp].astype(BF16)
            v = qkv[:, :, off + 2 * hp:off + 3 * hp].astype(BF16)
            s = jax.lax.dot_general(q, k, (((2,), (2,)), ((0,), (0,))),
                                    preferred_element_type=F32) * scale
            s = jax.nn.softmax(s, axis=-1)
            o = jax.lax.dot_general(s.astype(BF16), v, (((2,), (1,)), ((0,), (0,))),
                                    preferred_element_type=F32)
            outs.append(o)
        o = jnp.concatenate(outs, axis=-1)
        y = y + (jnp.dot(o.reshape(S * N, heads * hp).astype(BF16), wo[d],
                         preferred_element_type=F32) + bo[d]).reshape(S, N, C)
        h2 = _ln_in(y, ln2g[d], ln2b[d])
        f = jnp.dot(h2.reshape(S * N, C).astype(BF16), wfc1[d],
                    preferred_element_type=F32) + bfc1[d]
        f = _silu(f).astype(BF16)
        y = y + (jnp.dot(f, wfc2[d], preferred_element_type=F32)
                 + bfc2[d]).reshape(S, N, C)
    yf = _ln_in(y, lnfg[0], lnfb[0])
    p = jnp.dot(yf.reshape(S * N, C).astype(BF16), wproj[...],
                preferred_element_type=F32)
    p = _silu(p * gproj[...] + bproj[...])
    o_ref[...] = p.reshape(S, N, Co)


def _fus_kernel(xp_ref, yp_ref, wfx_ref, wfy_ref, g_ref, b_ref, o_ref,
                *, rb, W, Cin, Co):
    i = pl.program_id(1)
    xr = xp_ref[0, pl.ds(i * rb, rb + 2)]
    yr = yp_ref[0, pl.ds(i * rb, rb + 2)]
    acc = jnp.zeros((rb * W, Co), F32)
    for dy in range(3):
        for dx in range(3):
            k = 3 * dy + dx
            acc = acc + jnp.dot(xr[dy:dy + rb, dx:dx + W, :].reshape(rb * W, Cin),
                                wfx_ref[k], preferred_element_type=F32)
            acc = acc + jnp.dot(yr[dy:dy + rb, dx:dx + W, :].reshape(rb * W, Co),
                                wfy_ref[k], preferred_element_type=F32)
    o = _silu(acc * g_ref[...] + b_ref[...])
    o_ref[0] = o.reshape(rb, W, Co)


def kernel(x, task_bh, params):
    p = params
    B, Cin, H, W = x.shape
    Cd = p['conv_1x1_w'].shape[0]
    Co = p['conv_proj_w'].shape[0]
    E = p['moe_w1'].shape[0]
    depth = p['wqkv'].shape[0]
    heads = 4
    hd = Cd // heads
    ph = pw = 8
    nph, npw = H // ph, W // pw
    pa, npat = ph * pw, nph * npw
    T = B * H * W

    xcl = jnp.transpose(x, (0, 2, 3, 1))
    xpad = jnp.pad(xcl, ((0, 0), (1, 1), (1, 1), (0, 0)))
    wk = jnp.transpose(p['conv_kxk_w'], (2, 3, 1, 0)).reshape(9, Cin, Cin)
    w1x1 = p['conv_1x1_w'][:, :, 0, 0].T
    wk_hi = wk.astype(BF16)
    wk_lo = (wk - wk_hi.astype(F32)).astype(BF16)
    w1_hi = w1x1.astype(BF16)
    w1_lo = (w1x1 - w1_hi.astype(F32)).astype(BF16)
    gk = p['conv_kxk_g'].reshape(1, Cin)
    bk = p['conv_kxk_b'].reshape(1, Cin)
    RB = 16
    import functools
    ya = pl.pallas_call(
        functools.partial(_stage_a_kernel, rb=RB, W=W, Cin=Cin, Cd=Cd),
        grid=(B, H // RB),
        in_specs=[
            pl.BlockSpec((1, H + 2, W + 2, Cin), lambda b, i: (b, 0, 0, 0)),
            pl.BlockSpec((9, Cin, Cin), lambda b, i: (0, 0, 0)),
            pl.BlockSpec((9, Cin, Cin), lambda b, i: (0, 0, 0)),
            pl.BlockSpec((1, Cin), lambda b, i: (0, 0)),
            pl.BlockSpec((1, Cin), lambda b, i: (0, 0)),
            pl.BlockSpec((Cin, Cd), lambda b, i: (0, 0)),
            pl.BlockSpec((Cin, Cd), lambda b, i: (0, 0)),
        ],
        out_specs=pl.BlockSpec((1, RB, W, Cd), lambda b, i: (b, i, 0, 0)),
        out_shape=jax.ShapeDtypeStruct((B, H, W, Cd), F32),
    )(xpad, wk_hi, wk_lo, gk, bk, w1_hi, w1_lo)

    yseq = (ya.reshape(B, nph, ph, npw, pw, Cd)
              .transpose(0, 2, 4, 1, 3, 5)
              .reshape(B * pa, npat, Cd))
    xt = yseq.reshape(T, Cd)

    wg = p['w_gate'][task_bh]
    wg_hi = wg.astype(BF16)
    wg_lo = (wg - wg_hi.astype(F32)).astype(BF16)
    TB_G = 2048
    nblk = T // TB_G
    gates, loss, imp, load = pl.pallas_call(
        functools.partial(_gating_kernel, E=E, nblk=nblk),
        grid=(nblk,),
        in_specs=[
            pl.BlockSpec((TB_G, Cd), lambda t: (t, 0)),
            pl.BlockSpec((Cd, E), lambda t: (0, 0)),
            pl.BlockSpec((Cd, E), lambda t: (0, 0)),
        ],
        out_specs=[
            pl.BlockSpec((TB_G, E), lambda t: (t, 0)),
            pl.BlockSpec((1, 1), lambda t: (0, 0)),
            pl.BlockSpec((1, E), lambda t: (0, 0)),
            pl.BlockSpec((1, E), lambda t: (0, 0)),
        ],
        out_shape=[
            jax.ShapeDtypeStruct((T, E), F32),
            jax.ShapeDtypeStruct((1, 1), F32),
            jax.ShapeDtypeStruct((1, E), F32),
            jax.ShapeDtypeStruct((1, E), F32),
        ],
    )(xt, wg_hi, wg_lo)

    TB_M = 2048
    EP = E // 2
    w1p = (p['moe_w1'].reshape(EP, 2, Cd, Cd).transpose(0, 2, 1, 3)
           .reshape(EP, Cd, 2 * Cd)).astype(BF16)
    b1p = p['moe_b1'].reshape(EP, 1, 2 * Cd)
    w2p = p['moe_w2'].reshape(EP, 2 * Cd, Cd).astype(BF16)
    b2p = p['moe_b2'].reshape(EP, 1, 2 * Cd)
    moe = pl.pallas_call(
        functools.partial(_moe_kernel, Cd=Cd),
        grid=(T // TB_M, EP),
        in_specs=[
            pl.BlockSpec((TB_M, Cd), lambda t, e: (t, 0)),
            pl.BlockSpec((TB_M, E), lambda t, e: (t, 0)),
            pl.BlockSpec((1, Cd, 2 * Cd), lambda t, e: (e, 0, 0)),
            pl.BlockSpec((1, 1, 2 * Cd), lambda t, e: (e, 0, 0)),
            pl.BlockSpec((1, 2 * Cd, Cd), lambda t, e: (e, 0, 0)),
            pl.BlockSpec((1, 1, 2 * Cd), lambda t, e: (e, 0, 0)),
        ],
        out_specs=pl.BlockSpec((TB_M, Cd), lambda t, e: (t, 0)),
        out_shape=jax.ShapeDtypeStruct((T, Cd), F32),
    )(xt.astype(BF16), gates, w1p, b1p, w2p, b2p)

    S = 32
    HP = 128
    wqkv_p = jnp.pad(
        p['wqkv'].reshape(depth, Cd, 3, heads, hd).transpose(0, 1, 3, 2, 4),
        ((0, 0), (0, 0), (0, 0), (0, 0), (0, HP - hd)),
    ).reshape(depth, Cd, heads * 3 * HP)
    bqkv_p = jnp.pad(
        p['bqkv'].reshape(depth, 3, heads, hd).transpose(0, 2, 1, 3),
        ((0, 0), (0, 0), (0, 0), (0, HP - hd)),
    ).reshape(depth, heads * 3 * HP)
    wo_p = jnp.pad(
        p['wo'].reshape(depth, heads, hd, Cd),
        ((0, 0), (0, 0), (0, HP - hd), (0, 0)),
    ).reshape(depth, heads * HP, Cd)
    wproj = p['conv_proj_w'][:, :, 0, 0].T
    ypseq = pl.pallas_call(
        functools.partial(_tf_kernel, S=S, N=npat, C=Cd, Co=Co,
                          heads=heads, hd=hd, depth=depth),
        grid=(B * pa // S,),
        in_specs=[
            pl.BlockSpec((S, npat, Cd), lambda i: (i, 0, 0)),
            pl.BlockSpec((depth, Cd), lambda i: (0, 0)),
            pl.BlockSpec((depth, Cd), lambda i: (0, 0)),
            pl.BlockSpec((depth, Cd, heads * 3 * HP), lambda i: (0, 0, 0)),
            pl.BlockSpec((depth, heads * 3 * HP), lambda i: (0, 0)),
            pl.BlockSpec((depth, heads * HP, Cd), lambda i: (0, 0, 0)),
            pl.BlockSpec((depth, Cd), lambda i: (0, 0)),
            pl.BlockSpec((depth, Cd), lambda i: (0, 0)),
            pl.BlockSpec((depth, Cd), lambda i: (0, 0)),
            pl.BlockSpec((depth, Cd, 2 * Cd), lambda i: (0, 0, 0)),
            pl.BlockSpec((depth, 2 * Cd), lambda i: (0, 0)),
            pl.BlockSpec((depth, 2 * Cd, Cd), lambda i: (0, 0, 0)),
            pl.BlockSpec((depth, Cd), lambda i: (0, 0)),
            pl.BlockSpec((1, Cd), lambda i: (0, 0)),
            pl.BlockSpec((1, Cd), lambda i: (0, 0)),
            pl.BlockSpec((Cd, Co), lambda i: (0, 0)),
            pl.BlockSpec((1, Co), lambda i: (0, 0)),
            pl.BlockSpec((1, Co), lambda i: (0, 0)),
        ],
        out_specs=pl.BlockSpec((S, npat, Co), lambda i: (i, 0, 0)),
        out_shape=jax.ShapeDtypeStruct((B * pa, npat, Co), F32),
    )(moe.reshape(B * pa, npat, Cd),
      p['ln1_g'], p['ln1_b'], wqkv_p.astype(BF16), bqkv_p,
      wo_p.astype(BF16), p['bo'], p['ln2_g'], p['ln2_b'],
      p['wfc1'].astype(BF16), p['bfc1'], p['wfc2'].astype(BF16), p['bfc2'],
      p['lnf_g'].reshape(1, Cd), p['lnf_b'].reshape(1, Cd),
      wproj.astype(BF16), p['conv_proj_g'].reshape(1, Co),
      p['conv_proj_b'].reshape(1, Co))

    yp = (ypseq.reshape(B, ph, pw, nph, npw, Co)
               .transpose(0, 3, 1, 4, 2, 5)
               .reshape(B, H, W, Co))
    yppad = jnp.pad(yp, ((0, 0), (1, 1), (1, 1), (0, 0))).astype(BF16)
    xpad_bf = xpad.astype(BF16)
    wfus = p['conv_fus_w']
    wfx = jnp.transpose(wfus[:, :Cin], (2, 3, 1, 0)).reshape(9, Cin, Co).astype(BF16)
    wfy = jnp.transpose(wfus[:, Cin:], (2, 3, 1, 0)).reshape(9, Co, Co).astype(BF16)
    out = pl.pallas_call(
        functools.partial(_fus_kernel, rb=RB, W=W, Cin=Cin, Co=Co),
        grid=(B, H // RB),
        in_specs=[
            pl.BlockSpec((1, H + 2, W + 2, Cin), lambda b, i: (b, 0, 0, 0)),
            pl.BlockSpec((1, H + 2, W + 2, Co), lambda b, i: (b, 0, 0, 0)),
            pl.BlockSpec((9, Cin, Co), lambda b, i: (0, 0, 0)),
            pl.BlockSpec((9, Co, Co), lambda b, i: (0, 0, 0)),
            pl.BlockSpec((1, Co), lambda b, i: (0, 0)),
            pl.BlockSpec((1, Co), lambda b, i: (0, 0)),
        ],
        out_specs=pl.BlockSpec((1, RB, W, Co), lambda b, i: (b, i, 0, 0)),
        out_shape=jax.ShapeDtypeStruct((B, H, W, Co), F32),
    )(xpad_bf, yppad, wfx, wfy,
      p['conv_fus_g'].reshape(1, Co), p['conv_fus_b'].reshape(1, Co))

    y_final = jnp.transpose(out, (0, 3, 1, 2))
    return y_final, loss.reshape(())


# speedup vs baseline: 3.9593x; 1.0844x over previous
import jax
import jax.numpy as jnp
from jax.experimental import pallas as pl
from jax.experimental.pallas import tpu as pltpu

HIGH = jax.lax.Precision.HIGHEST
F32 = jnp.float32
BF16 = jnp.bfloat16


def _silu(x):
    return x * jax.nn.sigmoid(x)


def _split_hi_lo(v):
    hi = v.astype(BF16)
    lo = (v - hi.astype(F32)).astype(BF16)
    return hi, lo


def _dot3(x, w_hi, w_lo):
    x_hi, x_lo = _split_hi_lo(x)
    acc = jnp.dot(x_hi, w_hi, preferred_element_type=F32)
    acc += jnp.dot(x_hi, w_lo, preferred_element_type=F32)
    acc += jnp.dot(x_lo, w_hi, preferred_element_type=F32)
    return acc


def _ln_in(x, g, b):
    m = jnp.mean(x, -1, keepdims=True)
    v = jnp.mean((x - m) ** 2, -1, keepdims=True)
    return (x - m) * jax.lax.rsqrt(v + 1e-5) * g + b


def _stage_a_kernel(xp_ref, wkh_ref, wkl_ref, g_ref, b_ref, w1h_ref, w1l_ref,
                    o_ref, o2_ref, *, rb, W, Cin, Cd):
    i = pl.program_id(1)
    rows = xp_ref[0, pl.ds(i * rb, rb + 2)]
    rows_hi, rows_lo = _split_hi_lo(rows)
    acc = jnp.zeros((rb * W, Cin), F32)
    for dy in range(3):
        for dx in range(3):
            k = 3 * dy + dx
            xh = rows_hi[dy:dy + rb, dx:dx + W, :].reshape(rb * W, Cin)
            xl = rows_lo[dy:dy + rb, dx:dx + W, :].reshape(rb * W, Cin)
            acc = acc + jnp.dot(xh, wkh_ref[k], preferred_element_type=F32)
            acc = acc + jnp.dot(xh, wkl_ref[k], preferred_element_type=F32)
            acc = acc + jnp.dot(xl, wkh_ref[k], preferred_element_type=F32)
    y = _silu(acc * g_ref[...] + b_ref[...])
    out = _dot3(y, w1h_ref[...], w1l_ref[...])
    out = out.reshape(rb // 8, 8, 8, 8, Cd).transpose(1, 3, 0, 2, 4)
    o_ref[0] = out
    o2_ref[0] = out.astype(BF16)


def _gating_kernel(x_ref, wgh_ref, wgl_ref, gates_ref, loss_ref, imp_ref,
                   load_ref, *, E, nblk):
    t = pl.program_id(0)
    x = x_ref[...]
    logits = _dot3(x, wgh_ref[...], wgl_ref[...])
    tb = logits.shape[0]
    iota = jax.lax.broadcasted_iota(jnp.int32, (tb, E), 1)
    m1 = jnp.max(logits, axis=1, keepdims=True)
    i1 = jnp.argmax(logits, axis=1)[:, None]
    masked = jnp.where(iota == i1, -jnp.inf, logits)
    m2 = jnp.max(masked, axis=1, keepdims=True)
    i2 = jnp.argmax(masked, axis=1)[:, None]
    w1s = jax.nn.sigmoid(m1 - m2)
    w2s = jax.nn.sigmoid(m2 - m1)
    g = jnp.where(iota == i1, w1s, 0.0) + jnp.where(iota == i2, w2s, 0.0)
    gates_ref[...] = g

    @pl.when(t == 0)
    def _():
        imp_ref[...] = jnp.zeros_like(imp_ref)
        load_ref[...] = jnp.zeros_like(load_ref)

    imp_ref[...] += jnp.sum(g, axis=0, keepdims=True)
    load_ref[...] += jnp.sum((g > 0).astype(F32), axis=0, keepdims=True)

    @pl.when(t == nblk - 1)
    def _():
        def cv2(v):
            m = jnp.mean(v)
            var = jnp.mean((v - m) ** 2)
            return var / (m * m + 1e-10)

        val = cv2(imp_ref[0]) + cv2(load_ref[0])
        loss_ref[...] = jnp.broadcast_to(val, (1, 1))


def _moe_kernel(x_ref, g_ref, w1_ref, b1_ref, w2_ref, b2_ref, o_ref, *, Cd):
    e = pl.program_id(1)
    x = x_ref[...]
    tb = x.shape[0]
    h = jnp.dot(x, w1_ref[0], preferred_element_type=F32) + b1_ref[0]
    h = jnp.maximum(h, 0.0)
    g8 = g_ref[...]
    iota = jax.lax.broadcasted_iota(jnp.int32, g8.shape, 1)
    ga = jnp.sum(jnp.where(iota == 2 * e, g8, 0.0), axis=1, keepdims=True)
    gb = jnp.sum(jnp.where(iota == 2 * e + 1, g8, 0.0), axis=1, keepdims=True)
    gh = jnp.concatenate(
        [jnp.broadcast_to(ga, (tb, Cd)), jnp.broadcast_to(gb, (tb, Cd))], axis=1)
    hg = (h * gh).astype(BF16)
    oe = jnp.dot(hg, w2_ref[0], preferred_element_type=F32)
    oe += ga * b2_ref[0, :, :Cd] + gb * b2_ref[0, :, Cd:]
    @pl.when(e == 0)
    def _():
        o_ref[...] = oe

    @pl.when(e > 0)
    def _():
        o_ref[...] += oe


def _tf_kernel(y_ref, ln1g, ln1b, wqkv, bqkv, wo, bo, ln2g, ln2b,
               wfc1, bfc1, wfc2, bfc2, lnfg, lnfb, wproj, gproj, bproj,
               o_ref, *, S, N, C, Co, heads, hd, depth):
    y = y_ref[...]
    scale = hd ** -0.5
    hp = 128
    for d in range(depth):
        h1 = _ln_in(y, ln1g[d], ln1b[d])
        qkv = (jnp.dot(h1.reshape(S * N, C).astype(BF16), wqkv[d],
                       preferred_element_type=F32)
               + bqkv[d]).reshape(S, N, 3 * heads * hp)
        outs = []
        for h in range(heads):
            off = h * 3 * hp
            q = qkv[:, :, off:off + hp].astype(BF16)
            k = qkv[:, :, off + hp:off + 2 * hp].astype(BF16)
            v = qkv[:, :, off + 2 * hp:off + 3 * hp].astype(BF16)
            s = jax.lax.dot_general(q, k, (((2,), (2,)), ((0,), (0,))),
                                    preferred_element_type=F32) * scale
            s = jax.nn.softmax(s, axis=-1)
            o = jax.lax.dot_general(s.astype(BF16), v, (((2,), (1,)), ((0,), (0,))),
                                    preferred_element_type=F32)
            outs.append(o)
        o = jnp.concatenate(outs, axis=-1)
        y = y + (jnp.dot(o.reshape(S * N, heads * hp).astype(BF16), wo[d],
                         preferred_element_type=F32) + bo[d]).reshape(S, N, C)
        h2 = _ln_in(y, ln2g[d], ln2b[d])
        f = jnp.dot(h2.reshape(S * N, C).astype(BF16), wfc1[d],
                    preferred_element_type=F32) + bfc1[d]
        f = _silu(f).astype(BF16)
        y = y + (jnp.dot(f, wfc2[d], preferred_element_type=F32)
                 + bfc2[d]).reshape(S, N, C)
    yf = _ln_in(y, lnfg[0], lnfb[0])
    p = jnp.dot(yf.reshape(S * N, C).astype(BF16), wproj[...],
                preferred_element_type=F32)
    p = _silu(p * gproj[...] + bproj[...])
    p = p.reshape(S // 8, 8, 8, 8, Co).transpose(2, 0, 3, 1, 4)
    o_ref[0] = p.astype(BF16)


def _fus_kernel(xp_ref, yp_ref, wfx_ref, wfy_ref, g_ref, b_ref, o_ref,
                *, rb, W, Cin, Co):
    i = pl.program_id(1)
    xr = xp_ref[0, pl.ds(i * rb, rb + 2)]
    yr = yp_ref[0, pl.ds(i * rb, rb + 2)]
    acc = jnp.zeros((rb * W, Co), F32)
    for dy in range(3):
        for dx in range(3):
            k = 3 * dy + dx
            acc = acc + jnp.dot(xr[dy:dy + rb, dx:dx + W, :].reshape(rb * W, Cin),
                                wfx_ref[k], preferred_element_type=F32)
            acc = acc + jnp.dot(yr[dy:dy + rb, dx:dx + W, :].reshape(rb * W, Co),
                                wfy_ref[k], preferred_element_type=F32)
    o = _silu(acc * g_ref[...] + b_ref[...])
    o_ref[0] = o.reshape(rb, W, Co)


def kernel(x, task_bh, params):
    p = params
    B, Cin, H, W = x.shape
    Cd = p['conv_1x1_w'].shape[0]
    Co = p['conv_proj_w'].shape[0]
    E = p['moe_w1'].shape[0]
    depth = p['wqkv'].shape[0]
    heads = 4
    hd = Cd // heads
    ph = pw = 8
    nph, npw = H // ph, W // pw
    pa, npat = ph * pw, nph * npw
    T = B * H * W

    xcl = jnp.transpose(x, (0, 2, 3, 1))
    xpad = jnp.pad(xcl, ((0, 0), (1, 1), (1, 1), (0, 0)))
    wk = jnp.transpose(p['conv_kxk_w'], (2, 3, 1, 0)).reshape(9, Cin, Cin)
    w1x1 = p['conv_1x1_w'][:, :, 0, 0].T
    wk_hi = wk.astype(BF16)
    wk_lo = (wk - wk_hi.astype(F32)).astype(BF16)
    w1_hi = w1x1.astype(BF16)
    w1_lo = (w1x1 - w1_hi.astype(F32)).astype(BF16)
    gk = p['conv_kxk_g'].reshape(1, Cin)
    bk = p['conv_kxk_b'].reshape(1, Cin)
    RB = 16
    import functools
    ya = pl.pallas_call(
        functools.partial(_stage_a_kernel, rb=RB, W=W, Cin=Cin, Cd=Cd),
        grid=(B, H // RB),
        in_specs=[
            pl.BlockSpec((1, H + 2, W + 2, Cin), lambda b, i: (b, 0, 0, 0)),
            pl.BlockSpec((9, Cin, Cin), lambda b, i: (0, 0, 0)),
            pl.BlockSpec((9, Cin, Cin), lambda b, i: (0, 0, 0)),
            pl.BlockSpec((1, Cin), lambda b, i: (0, 0)),
            pl.BlockSpec((1, Cin), lambda b, i: (0, 0)),
            pl.BlockSpec((Cin, Cd), lambda b, i: (0, 0)),
            pl.BlockSpec((Cin, Cd), lambda b, i: (0, 0)),
        ],
        out_specs=[
            pl.BlockSpec((1, ph, pw, RB // 8, npw, Cd),
                         lambda b, i: (b, 0, 0, i, 0, 0)),
            pl.BlockSpec((1, ph, pw, RB // 8, npw, Cd),
                         lambda b, i: (b, 0, 0, i, 0, 0)),
        ],
        out_shape=[
            jax.ShapeDtypeStruct((B, ph, pw, nph, npw, Cd), F32),
            jax.ShapeDtypeStruct((B, ph, pw, nph, npw, Cd), BF16),
        ],
    )(xpad, wk_hi, wk_lo, gk, bk, w1_hi, w1_lo)
    ya, ya_bf = ya

    xt = ya.reshape(T, Cd)
    xt_bf = ya_bf.reshape(T, Cd)

    wg = p['w_gate'][task_bh]
    wg_hi = wg.astype(BF16)
    wg_lo = (wg - wg_hi.astype(F32)).astype(BF16)
    TB_G = 2048
    nblk = T // TB_G
    gates, loss, imp, load = pl.pallas_call(
        functools.partial(_gating_kernel, E=E, nblk=nblk),
        grid=(nblk,),
        in_specs=[
            pl.BlockSpec((TB_G, Cd), lambda t: (t, 0)),
            pl.BlockSpec((Cd, E), lambda t: (0, 0)),
            pl.BlockSpec((Cd, E), lambda t: (0, 0)),
        ],
        out_specs=[
            pl.BlockSpec((TB_G, E), lambda t: (t, 0)),
            pl.BlockSpec((1, 1), lambda t: (0, 0)),
            pl.BlockSpec((1, E), lambda t: (0, 0)),
            pl.BlockSpec((1, E), lambda t: (0, 0)),
        ],
        out_shape=[
            jax.ShapeDtypeStruct((T, E), F32),
            jax.ShapeDtypeStruct((1, 1), F32),
            jax.ShapeDtypeStruct((1, E), F32),
            jax.ShapeDtypeStruct((1, E), F32),
        ],
    )(xt, wg_hi, wg_lo)

    TB_M = 2048
    EP = E // 2
    w1p = (p['moe_w1'].reshape(EP, 2, Cd, Cd).transpose(0, 2, 1, 3)
           .reshape(EP, Cd, 2 * Cd)).astype(BF16)
    b1p = p['moe_b1'].reshape(EP, 1, 2 * Cd)
    w2p = p['moe_w2'].reshape(EP, 2 * Cd, Cd).astype(BF16)
    b2p = p['moe_b2'].reshape(EP, 1, 2 * Cd)
    moe = pl.pallas_call(
        functools.partial(_moe_kernel, Cd=Cd),
        grid=(T // TB_M, EP),
        in_specs=[
            pl.BlockSpec((TB_M, Cd), lambda t, e: (t, 0)),
            pl.BlockSpec((TB_M, E), lambda t, e: (t, 0)),
            pl.BlockSpec((1, Cd, 2 * Cd), lambda t, e: (e, 0, 0)),
            pl.BlockSpec((1, 1, 2 * Cd), lambda t, e: (e, 0, 0)),
            pl.BlockSpec((1, 2 * Cd, Cd), lambda t, e: (e, 0, 0)),
            pl.BlockSpec((1, 1, 2 * Cd), lambda t, e: (e, 0, 0)),
        ],
        out_specs=pl.BlockSpec((TB_M, Cd), lambda t, e: (t, 0)),
        out_shape=jax.ShapeDtypeStruct((T, Cd), F32),
    )(xt_bf, gates, w1p, b1p, w2p, b2p)

    S = 32
    HP = 128
    wqkv_p = jnp.pad(
        p['wqkv'].reshape(depth, Cd, 3, heads, hd).transpose(0, 1, 3, 2, 4),
        ((0, 0), (0, 0), (0, 0), (0, 0), (0, HP - hd)),
    ).reshape(depth, Cd, heads * 3 * HP)
    bqkv_p = jnp.pad(
        p['bqkv'].reshape(depth, 3, heads, hd).transpose(0, 2, 1, 3),
        ((0, 0), (0, 0), (0, 0), (0, HP - hd)),
    ).reshape(depth, heads * 3 * HP)
    wo_p = jnp.pad(
        p['wo'].reshape(depth, heads, hd, Cd),
        ((0, 0), (0, 0), (0, HP - hd), (0, 0)),
    ).reshape(depth, heads * HP, Cd)
    wproj = p['conv_proj_w'][:, :, 0, 0].T
    ypseq = pl.pallas_call(
        functools.partial(_tf_kernel, S=S, N=npat, C=Cd, Co=Co,
                          heads=heads, hd=hd, depth=depth),
        grid=(B * pa // S,),
        in_specs=[
            pl.BlockSpec((S, npat, Cd), lambda i: (i, 0, 0)),
            pl.BlockSpec((depth, Cd), lambda i: (0, 0)),
            pl.BlockSpec((depth, Cd), lambda i: (0, 0)),
            pl.BlockSpec((depth, Cd, heads * 3 * HP), lambda i: (0, 0, 0)),
            pl.BlockSpec((depth, heads * 3 * HP), lambda i: (0, 0)),
            pl.BlockSpec((depth, heads * HP, Cd), lambda i: (0, 0, 0)),
            pl.BlockSpec((depth, Cd), lambda i: (0, 0)),
            pl.BlockSpec((depth, Cd), lambda i: (0, 0)),
            pl.BlockSpec((depth, Cd), lambda i: (0, 0)),
            pl.BlockSpec((depth, Cd, 2 * Cd), lambda i: (0, 0, 0)),
            pl.BlockSpec((depth, 2 * Cd), lambda i: (0, 0)),
            pl.BlockSpec((depth, 2 * Cd, Cd), lambda i: (0, 0, 0)),
            pl.BlockSpec((depth, Cd), lambda i: (0, 0)),
            pl.BlockSpec((1, Cd), lambda i: (0, 0)),
            pl.BlockSpec((1, Cd), lambda i: (0, 0)),
            pl.BlockSpec((Cd, Co), lambda i: (0, 0)),
            pl.BlockSpec((1, Co), lambda i: (0, 0)),
            pl.BlockSpec((1, Co), lambda i: (0, 0)),
        ],
        out_specs=pl.BlockSpec((1, nph, S // ph, npw, pw, Co),
                               lambda i: (i // 2, 0, i % 2, 0, 0, 0)),
        out_shape=jax.ShapeDtypeStruct((B, nph, ph, npw, pw, Co), BF16),
    )(moe.reshape(B * pa, npat, Cd),
      p['ln1_g'], p['ln1_b'], wqkv_p.astype(BF16), bqkv_p,
      wo_p.astype(BF16), p['bo'], p['ln2_g'], p['ln2_b'],
      p['wfc1'].astype(BF16), p['bfc1'], p['wfc2'].astype(BF16), p['bfc2'],
      p['lnf_g'].reshape(1, Cd), p['lnf_b'].reshape(1, Cd),
      wproj.astype(BF16), p['conv_proj_g'].reshape(1, Co),
      p['conv_proj_b'].reshape(1, Co))

    yp = ypseq.reshape(B, H, W, Co)
    yppad = jnp.pad(yp, ((0, 0), (1, 1), (1, 1), (0, 0)))
    xpad_bf = xpad.astype(BF16)
    wfus = p['conv_fus_w']
    wfx = jnp.transpose(wfus[:, :Cin], (2, 3, 1, 0)).reshape(9, Cin, Co).astype(BF16)
    wfy = jnp.transpose(wfus[:, Cin:], (2, 3, 1, 0)).reshape(9, Co, Co).astype(BF16)
    out = pl.pallas_call(
        functools.partial(_fus_kernel, rb=RB, W=W, Cin=Cin, Co=Co),
        grid=(B, H // RB),
        in_specs=[
            pl.BlockSpec((1, H + 2, W + 2, Cin), lambda b, i: (b, 0, 0, 0)),
            pl.BlockSpec((1, H + 2, W + 2, Co), lambda b, i: (b, 0, 0, 0)),
            pl.BlockSpec((9, Cin, Co), lambda b, i: (0, 0, 0)),
            pl.BlockSpec((9, Co, Co), lambda b, i: (0, 0, 0)),
            pl.BlockSpec((1, Co), lambda b, i: (0, 0)),
            pl.BlockSpec((1, Co), lambda b, i: (0, 0)),
        ],
        out_specs=pl.BlockSpec((1, RB, W, Co), lambda b, i: (b, i, 0, 0)),
        out_shape=jax.ShapeDtypeStruct((B, H, W, Co), F32),
    )(xpad_bf, yppad, wfx, wfy,
      p['conv_fus_g'].reshape(1, Co), p['conv_fus_b'].reshape(1, Co))

    y_final = jnp.transpose(out, (0, 3, 1, 2))
    return y_final, loss.reshape(())


# speedup vs baseline: 3.9704x; 1.0028x over previous
import jax
import jax.numpy as jnp
from jax.experimental import pallas as pl
from jax.experimental.pallas import tpu as pltpu

HIGH = jax.lax.Precision.HIGHEST
F32 = jnp.float32
BF16 = jnp.bfloat16


def _silu(x):
    return x * jax.nn.sigmoid(x)


def _split_hi_lo(v):
    hi = v.astype(BF16)
    lo = (v - hi.astype(F32)).astype(BF16)
    return hi, lo


def _dot3(x, w_hi, w_lo):
    x_hi, x_lo = _split_hi_lo(x)
    acc = jnp.dot(x_hi, w_hi, preferred_element_type=F32)
    acc += jnp.dot(x_hi, w_lo, preferred_element_type=F32)
    acc += jnp.dot(x_lo, w_hi, preferred_element_type=F32)
    return acc


def _ln_in(x, g, b):
    m = jnp.mean(x, -1, keepdims=True)
    v = jnp.mean((x - m) ** 2, -1, keepdims=True)
    return (x - m) * jax.lax.rsqrt(v + 1e-5) * g + b


def _stage_a_kernel(xp_ref, wkh_ref, wkl_ref, g_ref, b_ref, w1h_ref, w1l_ref,
                    o_ref, o2_ref, *, rb, W, Cin, Cd):
    i = pl.program_id(1)
    rows = xp_ref[0, pl.ds(i * rb, rb + 2)]
    rows_hi, rows_lo = _split_hi_lo(rows)
    acc = jnp.zeros((rb * W, Cin), F32)
    for dy in range(3):
        for dx in range(3):
            k = 3 * dy + dx
            xh = rows_hi[dy:dy + rb, dx:dx + W, :].reshape(rb * W, Cin)
            xl = rows_lo[dy:dy + rb, dx:dx + W, :].reshape(rb * W, Cin)
            acc = acc + jnp.dot(xh, wkh_ref[k], preferred_element_type=F32)
            acc = acc + jnp.dot(xh, wkl_ref[k], preferred_element_type=F32)
            acc = acc + jnp.dot(xl, wkh_ref[k], preferred_element_type=F32)
    y = _silu(acc * g_ref[...] + b_ref[...])
    out = _dot3(y, w1h_ref[...], w1l_ref[...])
    out = out.reshape(rb // 8, 8, 8, 8, Cd).transpose(1, 3, 0, 2, 4)
    o_ref[0] = out
    o2_ref[0] = out.astype(BF16)


def _gating_kernel(x_ref, wgh_ref, wgl_ref, gates_ref, loss_ref, imp_ref,
                   load_ref, *, E, nblk):
    t = pl.program_id(0)
    x = x_ref[...]
    logits = _dot3(x, wgh_ref[...], wgl_ref[...])
    tb = logits.shape[0]
    iota = jax.lax.broadcasted_iota(jnp.int32, (tb, E), 1)
    m1 = jnp.max(logits, axis=1, keepdims=True)
    i1 = jnp.argmax(logits, axis=1)[:, None]
    masked = jnp.where(iota == i1, -jnp.inf, logits)
    m2 = jnp.max(masked, axis=1, keepdims=True)
    i2 = jnp.argmax(masked, axis=1)[:, None]
    w1s = jax.nn.sigmoid(m1 - m2)
    w2s = jax.nn.sigmoid(m2 - m1)
    g = jnp.where(iota == i1, w1s, 0.0) + jnp.where(iota == i2, w2s, 0.0)
    gates_ref[...] = g

    @pl.when(t == 0)
    def _():
        imp_ref[...] = jnp.zeros_like(imp_ref)
        load_ref[...] = jnp.zeros_like(load_ref)

    imp_ref[...] += jnp.sum(g, axis=0, keepdims=True)
    load_ref[...] += jnp.sum((g > 0).astype(F32), axis=0, keepdims=True)

    @pl.when(t == nblk - 1)
    def _():
        def cv2(v):
            m = jnp.mean(v)
            var = jnp.mean((v - m) ** 2)
            return var / (m * m + 1e-10)

        val = cv2(imp_ref[0]) + cv2(load_ref[0])
        loss_ref[...] = jnp.broadcast_to(val, (1, 1))


def _moe_kernel(x_ref, g_ref, w1_ref, b1_ref, w2_ref, b2_ref, o_ref, *, Cd):
    e = pl.program_id(1)
    x = x_ref[...]
    tb = x.shape[0]
    h = jnp.dot(x, w1_ref[0], preferred_element_type=F32) + b1_ref[0]
    h = jnp.maximum(h, 0.0)
    g8 = g_ref[...]
    iota = jax.lax.broadcasted_iota(jnp.int32, g8.shape, 1)
    ga = jnp.sum(jnp.where(iota == 2 * e, g8, 0.0), axis=1, keepdims=True)
    gb = jnp.sum(jnp.where(iota == 2 * e + 1, g8, 0.0), axis=1, keepdims=True)
    gh = jnp.concatenate(
        [jnp.broadcast_to(ga, (tb, Cd)), jnp.broadcast_to(gb, (tb, Cd))], axis=1)
    hg = (h * gh).astype(BF16)
    oe = jnp.dot(hg, w2_ref[0], preferred_element_type=F32)
    oe += ga * b2_ref[0, :, :Cd] + gb * b2_ref[0, :, Cd:]
    @pl.when(e == 0)
    def _():
        o_ref[...] = oe

    @pl.when(e > 0)
    def _():
        o_ref[...] += oe


def _tf_kernel(y_ref, ln1g, ln1b, wqkv, bqkv, wo, bo, ln2g, ln2b,
               wfc1, bfc1, wfc2, bfc2, lnfg, lnfb, wproj, gproj, bproj,
               o_ref, *, S, N, C, Co, heads, hd, depth):
    y = y_ref[...]
    scale = hd ** -0.5
    hp = 128
    for d in range(depth):
        h1 = _ln_in(y, ln1g[d], ln1b[d])
        qkv = (jnp.dot(h1.reshape(S * N, C).astype(BF16), wqkv[d],
                       preferred_element_type=F32)
               + bqkv[d]).reshape(S, N, 3 * heads * hp)
        outs = []
        for h in range(heads):
            off = h * 3 * hp
            q = qkv[:, :, off:off + hp].astype(BF16)
            k = qkv[:, :, off + hp:off + 2 * hp].astype(BF16)
            v = qkv[:, :, off + 2 * hp:off + 3 * hp].astype(BF16)
            s = jax.lax.dot_general(q, k, (((2,), (2,)), ((0,), (0,))),
                                    preferred_element_type=F32) * scale
            s = jax.nn.softmax(s, axis=-1)
            o = jax.lax.dot_general(s.astype(BF16), v, (((2,), (1,)), ((0,), (0,))),
                                    preferred_element_type=F32)
            outs.append(o)
        o = jnp.concatenate(outs, axis=-1)
        y = y + (jnp.dot(o.reshape(S * N, heads * hp).astype(BF16), wo[d],
                         preferred_element_type=F32) + bo[d]).reshape(S, N, C)
        h2 = _ln_in(y, ln2g[d], ln2b[d])
        f = jnp.dot(h2.reshape(S * N, C).astype(BF16), wfc1[d],
                    preferred_element_type=F32) + bfc1[d]
        f = _silu(f).astype(BF16)
        y = y + (jnp.dot(f, wfc2[d], preferred_element_type=F32)
                 + bfc2[d]).reshape(S, N, C)
    yf = _ln_in(y, lnfg[0], lnfb[0])
    p = jnp.dot(yf.reshape(S * N, C).astype(BF16), wproj[...],
                preferred_element_type=F32)
    p = _silu(p * gproj[...] + bproj[...])
    p = p.reshape(S // 8, 8, 8, 8, Co).transpose(2, 0, 3, 1, 4)
    o_ref[0] = p.astype(BF16)


def _fus_kernel(xp_ref, yp_ref, wfx_ref, wfy_ref, g_ref, b_ref, o_ref,
                *, rb, W, Cin, Co):
    i = pl.program_id(1)
    xr = xp_ref[0, pl.ds(i * rb, rb + 2)]
    yr = yp_ref[0, pl.ds(i * rb, rb + 2)]
    acc = jnp.zeros((rb * W, Co), F32)
    for dy in range(3):
        for dx in range(3):
            k = 3 * dy + dx
            acc = acc + jnp.dot(xr[dy:dy + rb, dx:dx + W, :].reshape(rb * W, Cin),
                                wfx_ref[k], preferred_element_type=F32)
            acc = acc + jnp.dot(yr[dy:dy + rb, dx:dx + W, :].reshape(rb * W, Co),
                                wfy_ref[k], preferred_element_type=F32)
    o = _silu(acc * g_ref[...] + b_ref[...])
    o_ref[0] = o.reshape(rb, W, Co)


def kernel(x, task_bh, params):
    p = params
    B, Cin, H, W = x.shape
    Cd = p['conv_1x1_w'].shape[0]
    Co = p['conv_proj_w'].shape[0]
    E = p['moe_w1'].shape[0]
    depth = p['wqkv'].shape[0]
    heads = 4
    hd = Cd // heads
    ph = pw = 8
    nph, npw = H // ph, W // pw
    pa, npat = ph * pw, nph * npw
    T = B * H * W

    xcl = jnp.transpose(x, (0, 2, 3, 1))
    xpad = jnp.pad(xcl, ((0, 0), (1, 1), (1, 1), (0, 0)))
    wk = jnp.transpose(p['conv_kxk_w'], (2, 3, 1, 0)).reshape(9, Cin, Cin)
    w1x1 = p['conv_1x1_w'][:, :, 0, 0].T
    wk_hi = wk.astype(BF16)
    wk_lo = (wk - wk_hi.astype(F32)).astype(BF16)
    w1_hi = w1x1.astype(BF16)
    w1_lo = (w1x1 - w1_hi.astype(F32)).astype(BF16)
    gk = p['conv_kxk_g'].reshape(1, Cin)
    bk = p['conv_kxk_b'].reshape(1, Cin)
    RB = 16
    import functools
    ya = pl.pallas_call(
        functools.partial(_stage_a_kernel, rb=RB, W=W, Cin=Cin, Cd=Cd),
        grid=(B, H // RB),
        in_specs=[
            pl.BlockSpec((1, H + 2, W + 2, Cin), lambda b, i: (b, 0, 0, 0)),
            pl.BlockSpec((9, Cin, Cin), lambda b, i: (0, 0, 0)),
            pl.BlockSpec((9, Cin, Cin), lambda b, i: (0, 0, 0)),
            pl.BlockSpec((1, Cin), lambda b, i: (0, 0)),
            pl.BlockSpec((1, Cin), lambda b, i: (0, 0)),
            pl.BlockSpec((Cin, Cd), lambda b, i: (0, 0)),
            pl.BlockSpec((Cin, Cd), lambda b, i: (0, 0)),
        ],
        out_specs=[
            pl.BlockSpec((1, ph, pw, RB // 8, npw, Cd),
                         lambda b, i: (b, 0, 0, i, 0, 0)),
            pl.BlockSpec((1, ph, pw, RB // 8, npw, Cd),
                         lambda b, i: (b, 0, 0, i, 0, 0)),
        ],
        out_shape=[
            jax.ShapeDtypeStruct((B, ph, pw, nph, npw, Cd), F32),
            jax.ShapeDtypeStruct((B, ph, pw, nph, npw, Cd), BF16),
        ],
    )(xpad, wk_hi, wk_lo, gk, bk, w1_hi, w1_lo)
    ya, ya_bf = ya

    xt = ya.reshape(T, Cd)
    xt_bf = ya_bf.reshape(T, Cd)

    wg = p['w_gate'][task_bh]
    wg_hi = wg.astype(BF16)
    wg_lo = (wg - wg_hi.astype(F32)).astype(BF16)
    TB_G = 2048
    nblk = T // TB_G
    gates, loss, imp, load = pl.pallas_call(
        functools.partial(_gating_kernel, E=E, nblk=nblk),
        grid=(nblk,),
        in_specs=[
            pl.BlockSpec((TB_G, Cd), lambda t: (t, 0)),
            pl.BlockSpec((Cd, E), lambda t: (0, 0)),
            pl.BlockSpec((Cd, E), lambda t: (0, 0)),
        ],
        out_specs=[
            pl.BlockSpec((TB_G, E), lambda t: (t, 0)),
            pl.BlockSpec((1, 1), lambda t: (0, 0)),
            pl.BlockSpec((1, E), lambda t: (0, 0)),
            pl.BlockSpec((1, E), lambda t: (0, 0)),
        ],
        out_shape=[
            jax.ShapeDtypeStruct((T, E), F32),
            jax.ShapeDtypeStruct((1, 1), F32),
            jax.ShapeDtypeStruct((1, E), F32),
            jax.ShapeDtypeStruct((1, E), F32),
        ],
    )(xt, wg_hi, wg_lo)

    TB_M = 4096
    EP = E // 2
    w1p = (p['moe_w1'].reshape(EP, 2, Cd, Cd).transpose(0, 2, 1, 3)
           .reshape(EP, Cd, 2 * Cd)).astype(BF16)
    b1p = p['moe_b1'].reshape(EP, 1, 2 * Cd)
    w2p = p['moe_w2'].reshape(EP, 2 * Cd, Cd).astype(BF16)
    b2p = p['moe_b2'].reshape(EP, 1, 2 * Cd)
    moe = pl.pallas_call(
        functools.partial(_moe_kernel, Cd=Cd),
        grid=(T // TB_M, EP),
        in_specs=[
            pl.BlockSpec((TB_M, Cd), lambda t, e: (t, 0)),
            pl.BlockSpec((TB_M, E), lambda t, e: (t, 0)),
            pl.BlockSpec((1, Cd, 2 * Cd), lambda t, e: (e, 0, 0)),
            pl.BlockSpec((1, 1, 2 * Cd), lambda t, e: (e, 0, 0)),
            pl.BlockSpec((1, 2 * Cd, Cd), lambda t, e: (e, 0, 0)),
            pl.BlockSpec((1, 1, 2 * Cd), lambda t, e: (e, 0, 0)),
        ],
        out_specs=pl.BlockSpec((TB_M, Cd), lambda t, e: (t, 0)),
        out_shape=jax.ShapeDtypeStruct((T, Cd), F32),
    )(xt_bf, gates, w1p, b1p, w2p, b2p)

    S = 32
    HP = 128
    wqkv_p = jnp.pad(
        p['wqkv'].reshape(depth, Cd, 3, heads, hd).transpose(0, 1, 3, 2, 4),
        ((0, 0), (0, 0), (0, 0), (0, 0), (0, HP - hd)),
    ).reshape(depth, Cd, heads * 3 * HP)
    bqkv_p = jnp.pad(
        p['bqkv'].reshape(depth, 3, heads, hd).transpose(0, 2, 1, 3),
        ((0, 0), (0, 0), (0, 0), (0, HP - hd)),
    ).reshape(depth, heads * 3 * HP)
    wo_p = jnp.pad(
        p['wo'].reshape(depth, heads, hd, Cd),
        ((0, 0), (0, 0), (0, HP - hd), (0, 0)),
    ).reshape(depth, heads * HP, Cd)
    wproj = p['conv_proj_w'][:, :, 0, 0].T
    ypseq = pl.pallas_call(
        functools.partial(_tf_kernel, S=S, N=npat, C=Cd, Co=Co,
                          heads=heads, hd=hd, depth=depth),
        grid=(B * pa // S,),
        in_specs=[
            pl.BlockSpec((S, npat, Cd), lambda i: (i, 0, 0)),
            pl.BlockSpec((depth, Cd), lambda i: (0, 0)),
            pl.BlockSpec((depth, Cd), lambda i: (0, 0)),
            pl.BlockSpec((depth, Cd, heads * 3 * HP), lambda i: (0, 0, 0)),
            pl.BlockSpec((depth, heads * 3 * HP), lambda i: (0, 0)),
            pl.BlockSpec((depth, heads * HP, Cd), lambda i: (0, 0, 0)),
            pl.BlockSpec((depth, Cd), lambda i: (0, 0)),
            pl.BlockSpec((depth, Cd), lambda i: (0, 0)),
            pl.BlockSpec((depth, Cd), lambda i: (0, 0)),
            pl.BlockSpec((depth, Cd, 2 * Cd), lambda i: (0, 0, 0)),
            pl.BlockSpec((depth, 2 * Cd), lambda i: (0, 0)),
            pl.BlockSpec((depth, 2 * Cd, Cd), lambda i: (0, 0, 0)),
            pl.BlockSpec((depth, Cd), lambda i: (0, 0)),
            pl.BlockSpec((1, Cd), lambda i: (0, 0)),
            pl.BlockSpec((1, Cd), lambda i: (0, 0)),
            pl.BlockSpec((Cd, Co), lambda i: (0, 0)),
            pl.BlockSpec((1, Co), lambda i: (0, 0)),
            pl.BlockSpec((1, Co), lambda i: (0, 0)),
        ],
        out_specs=pl.BlockSpec((1, nph, S // ph, npw, pw, Co),
                               lambda i: (i // 2, 0, i % 2, 0, 0, 0)),
        out_shape=jax.ShapeDtypeStruct((B, nph, ph, npw, pw, Co), BF16),
    )(moe.reshape(B * pa, npat, Cd),
      p['ln1_g'], p['ln1_b'], wqkv_p.astype(BF16), bqkv_p,
      wo_p.astype(BF16), p['bo'], p['ln2_g'], p['ln2_b'],
      p['wfc1'].astype(BF16), p['bfc1'], p['wfc2'].astype(BF16), p['bfc2'],
      p['lnf_g'].reshape(1, Cd), p['lnf_b'].reshape(1, Cd),
      wproj.astype(BF16), p['conv_proj_g'].reshape(1, Co),
      p['conv_proj_b'].reshape(1, Co))

    yp = ypseq.reshape(B, H, W, Co)
    yppad = jnp.pad(yp, ((0, 0), (1, 1), (1, 1), (0, 0)))
    xpad_bf = xpad.astype(BF16)
    wfus = p['conv_fus_w']
    wfx = jnp.transpose(wfus[:, :Cin], (2, 3, 1, 0)).reshape(9, Cin, Co).astype(BF16)
    wfy = jnp.transpose(wfus[:, Cin:], (2, 3, 1, 0)).reshape(9, Co, Co).astype(BF16)
    out = pl.pallas_call(
        functools.partial(_fus_kernel, rb=RB, W=W, Cin=Cin, Co=Co),
        grid=(B, H // RB),
        in_specs=[
            pl.BlockSpec((1, H + 2, W + 2, Cin), lambda b, i: (b, 0, 0, 0)),
            pl.BlockSpec((1, H + 2, W + 2, Co), lambda b, i: (b, 0, 0, 0)),
            pl.BlockSpec((9, Cin, Co), lambda b, i: (0, 0, 0)),
            pl.BlockSpec((9, Co, Co), lambda b, i: (0, 0, 0)),
            pl.BlockSpec((1, Co), lambda b, i: (0, 0)),
            pl.BlockSpec((1, Co), lambda b, i: (0, 0)),
        ],
        out_specs=pl.BlockSpec((1, RB, W, Co), lambda b, i: (b, i, 0, 0)),
        out_shape=jax.ShapeDtypeStruct((B, H, W, Co), F32),
    )(xpad_bf, yppad, wfx, wfy,
      p['conv_fus_g'].reshape(1, Co), p['conv_fus_b'].reshape(1, Co))

    y_final = jnp.transpose(out, (0, 3, 1, 2))
    return y_final, loss.reshape(())
